# Initial kernel scaffold; baseline (speedup 1.0000x reference)
import jax
import jax.numpy as jnp
from jax.experimental import pallas as pl


def kernel(x, edge_index, edge_attr, batch, emb, W1, b1, W2, b2, Wm1, bm1, Wm2, bm2):
    raise NotImplementedError("write your pallas kernel here")


# speedup vs baseline: 2.3526x; 2.3526x over previous
import functools

import jax
import jax.numpy as jnp
from jax import lax
from jax.experimental import pallas as pl
from jax.experimental.pallas import tpu as pltpu

N = 10000
E = 320000
G = 128
VOCAB = 100000
EMB = 128
HID = 128

NP_ = 10240
BLK = 256
NBLK = NP_ // BLK


def _prep_body(h_ref, m_ref, s_ref, w_ref, o_ref):
    h = h_ref[...] * m_ref[...]
    o_ref[...] = s_ref[...] * jnp.dot(h, w_ref[...],
                                      preferred_element_type=jnp.float32)


def _prep_layer(h, m, s, w):
    return pl.pallas_call(
        _prep_body,
        grid=(NBLK,),
        in_specs=[
            pl.BlockSpec((BLK, HID), lambda i: (i, 0)),
            pl.BlockSpec((BLK, 1), lambda i: (i, 0)),
            pl.BlockSpec((BLK, 1), lambda i: (i, 0)),
            pl.BlockSpec((HID, HID), lambda i: (0, 0)),
        ],
        out_specs=pl.BlockSpec((BLK, HID), lambda i: (i, 0)),
        out_shape=jax.ShapeDtypeStruct((NP_, HID), jnp.float32),
    )(h, m, s, w)


def _combine_body(acc_ref, hs_ref, s_ref, b_ref, w_ref, o_ref):
    t = jax.nn.relu(s_ref[...] * (acc_ref[...] + hs_ref[...]) + b_ref[...])
    o_ref[...] = s_ref[...] * jnp.dot(t, w_ref[...],
                                      preferred_element_type=jnp.float32)


def _combine_layer(acc, hs, s, b, w):
    return pl.pallas_call(
        _combine_body,
        grid=(NBLK,),
        in_specs=[
            pl.BlockSpec((BLK, HID), lambda i: (i, 0)),
            pl.BlockSpec((BLK, HID), lambda i: (i, 0)),
            pl.BlockSpec((BLK, 1), lambda i: (i, 0)),
            pl.BlockSpec((1, HID), lambda i: (0, 0)),
            pl.BlockSpec((HID, HID), lambda i: (0, 0)),
        ],
        out_specs=pl.BlockSpec((BLK, HID), lambda i: (i, 0)),
        out_shape=jax.ShapeDtypeStruct((NP_, HID), jnp.float32),
    )(acc, hs, s, b, w)


def _final_body(acc_ref, hs_ref, s_ref, b_ref, batch_ref,
                wm1_ref, bm1_ref, wm2_ref, bm2_ref, o_ref,
                gsum_ref, cnt_ref):
    i = pl.program_id(0)

    @pl.when(i == 0)
    def _():
        gsum_ref[...] = jnp.zeros_like(gsum_ref)
        cnt_ref[...] = jnp.zeros_like(cnt_ref)

    h = jax.nn.relu(s_ref[...] * (acc_ref[...] + hs_ref[...]) + b_ref[...])
    seg = batch_ref[...]
    gid = lax.broadcasted_iota(jnp.int32, (G, BLK), 0)
    oh = (seg == gid).astype(jnp.float32)
    gsum_ref[...] += jnp.dot(oh, h, preferred_element_type=jnp.float32)
    cnt_ref[...] += jnp.sum(oh, axis=1, keepdims=True)

    @pl.when(i == NBLK - 1)
    def _():
        g = gsum_ref[...] / jnp.maximum(cnt_ref[...], 1.0)
        a = jax.nn.relu(jnp.dot(g, wm1_ref[...],
                                preferred_element_type=jnp.float32)
                        + bm1_ref[...])
        o = jnp.dot(a, wm2_ref[...],
                    preferred_element_type=jnp.float32) + bm2_ref[...]
        o = o - jnp.max(o, axis=1, keepdims=True)
        eo = jnp.exp(o)
        o_ref[...] = eo / jnp.sum(eo, axis=1, keepdims=True)


def _final_layer(acc, hs, s, b, batch3d, wm1, bm1, wm2, bm2):
    return pl.pallas_call(
        _final_body,
        grid=(NBLK,),
        in_specs=[
            pl.BlockSpec((BLK, HID), lambda i: (i, 0)),
            pl.BlockSpec((BLK, HID), lambda i: (i, 0)),
            pl.BlockSpec((BLK, 1), lambda i: (i, 0)),
            pl.BlockSpec((1, HID), lambda i: (0, 0)),
            pl.BlockSpec((None, 1, BLK), lambda i: (i, 0, 0)),
            pl.BlockSpec((HID, HID // 2), lambda i: (0, 0)),
            pl.BlockSpec((1, HID // 2), lambda i: (0, 0)),
            pl.BlockSpec((HID // 2, 2), lambda i: (0, 0)),
            pl.BlockSpec((1, 2), lambda i: (0, 0)),
        ],
        out_specs=pl.BlockSpec((G, 2), lambda i: (0, 0)),
        out_shape=jax.ShapeDtypeStruct((G, 2), jnp.float32),
        scratch_shapes=[
            pltpu.VMEM((G, HID), jnp.float32),
            pltpu.VMEM((G, 1), jnp.float32),
        ],
    )(acc, hs, s, b, batch3d, wm1, bm1, wm2, bm2)


def kernel(x, edge_index, edge_attr, batch, emb, W1, b1, W2, b2,
           Wm1, bm1, Wm2, bm2):
    src, dst = edge_index[0], edge_index[1]

    h0 = emb[x]
    deg = jnp.zeros((N,), jnp.float32).at[dst].add(edge_attr) + 1.0

    h0p = jnp.pad(h0, ((0, NP_ - N), (0, 0)))
    xp = jnp.pad(x, (0, NP_ - N)).reshape(NP_, 1)
    degp = jnp.pad(deg, (0, NP_ - N), constant_values=1.0)
    dinv = lax.rsqrt(degp).reshape(NP_, 1)
    mask = (xp != 0).astype(jnp.float32)

    hs1 = _prep_layer(h0p, mask, dinv, W1)

    acc1 = jnp.zeros((NP_, HID), jnp.float32).at[dst].add(
        edge_attr[:, None] * hs1[src])

    hs2 = _combine_layer(acc1, hs1, dinv, b1.reshape(1, HID), W2)

    acc2 = jnp.zeros((NP_, HID), jnp.float32).at[dst].add(
        edge_attr[:, None] * hs2[src])

    batchp = jnp.pad(batch, (0, NP_ - N), constant_values=-1)
    batch3d = batchp.reshape(NBLK, 1, BLK)

    return _final_layer(acc2, hs2, dinv, b2.reshape(1, HID), batch3d,
                        Wm1, bm1.reshape(1, HID // 2), Wm2,
                        bm2.reshape(1, 2))


# speedup vs baseline: 7.2671x; 3.0890x over previous
import functools

import jax
import jax.numpy as jnp
from jax import lax
from jax.experimental import pallas as pl
from jax.experimental.pallas import tpu as pltpu
from jax.experimental.pallas import tpu_sc as plsc

N = 10000
E = 320000
G = 128
VOCAB = 100000
EMB = 128
HID = 128

NP_ = 10240
BLK = 256
NBLK = NP_ // BLK

NSC = 2
NTILE = 16
ECHUNK = 128
NCHUNK = 79
EP = NSC * NTILE * NCHUNK * ECHUNK
EPT = NCHUNK * ECHUNK
RPT = NP_ // NTILE


def _agg_body(hs_hbm, src_hbm, dst_hbm, ew_hbm, out_hbm,
              srcv, dstv, ewv, rows_v, acc_sh, sem):
    c = lax.axis_index("c")
    s = lax.axis_index("s")

    def zero_body(t, _):
        r = t // 8
        j = t % 8
        rows_v[r, pl.ds(j * 16, 16)] = jnp.zeros((16,), jnp.float32)
        return 0
    lax.fori_loop(0, ECHUNK * 8, zero_body, 0, unroll=8)
    for k in range(RPT // ECHUNK):
        pltpu.sync_copy(rows_v, acc_sh.at[pl.ds(s * RPT + k * ECHUNK, ECHUNK)])
    plsc.subcore_barrier()

    tile_base = (c * NTILE + s) * EPT

    def chunk_body(i, _):
        base = tile_base + i * ECHUNK
        pltpu.sync_copy(src_hbm.at[pl.ds(base, ECHUNK)], srcv)
        pltpu.sync_copy(dst_hbm.at[pl.ds(base, ECHUNK)], dstv)
        pltpu.sync_copy(ew_hbm.at[pl.ds(base, ECHUNK)], ewv)
        pltpu.async_copy(hs_hbm.at[srcv], rows_v, sem).wait()

        def scale_body(g, _):
            ewg = ewv[pl.ds(g * 16, 16)]
            for l in range(16):
                spl = jnp.full((16,), ewg[l], jnp.float32)
                e = g * 16 + l
                for j in range(8):
                    rows_v[e, pl.ds(j * 16, 16)] = \
                        rows_v[e, pl.ds(j * 16, 16)] * spl
            return 0
        lax.fori_loop(0, ECHUNK // 16, scale_body, 0)
        pltpu.sync_copy(rows_v, acc_sh.at[dstv], add=True)
        return 0
    lax.fori_loop(0, NCHUNK, chunk_body, 0)

    plsc.subcore_barrier()
    pltpu.sync_copy(acc_sh.at[pl.ds(s * RPT, RPT)],
                    out_hbm.at[c, pl.ds(s * RPT, RPT)])


def _sc_aggregate(hs, srcp, dstp, ewp):
    mesh = plsc.VectorSubcoreMesh(core_axis_name="c", subcore_axis_name="s")
    f = pl.kernel(
        _agg_body,
        out_type=jax.ShapeDtypeStruct((NSC, NP_, HID), jnp.float32),
        mesh=mesh,
        scratch_types=[
            pltpu.VMEM((ECHUNK,), jnp.int32),
            pltpu.VMEM((ECHUNK,), jnp.int32),
            pltpu.VMEM((ECHUNK,), jnp.float32),
            pltpu.VMEM((ECHUNK, HID), jnp.float32),
            pltpu.VMEM_SHARED((NP_, HID), jnp.float32),
            pltpu.SemaphoreType.DMA,
        ],
    )
    return f(hs, srcp, dstp, ewp)


XGP = NSC * NTILE * 3 * ECHUNK


def _gd_body(emb_hbm, xg_hbm, dst_hbm, ew_hbm, h0_hbm, deg_hbm,
             idxv, rows_v, dstv, ewv, zv, deg_sh, sem):
    c = lax.axis_index("c")
    s = lax.axis_index("s")
    wid = c * NTILE + s

    def zero_body(t, _):
        zv[pl.ds(t * 16, 16)] = jnp.zeros((16,), jnp.float32)
        return 0
    lax.fori_loop(0, RPT // 16, zero_body, 0, unroll=8)
    pltpu.sync_copy(zv, deg_sh.at[pl.ds(s * RPT, RPT)])
    plsc.subcore_barrier()

    for k in range(3):
        base = (wid * 3 + k) * ECHUNK
        pltpu.sync_copy(xg_hbm.at[pl.ds(base, ECHUNK)], idxv)
        pltpu.async_copy(emb_hbm.at[idxv], rows_v, sem).wait()
        pltpu.sync_copy(rows_v, h0_hbm.at[pl.ds(base, ECHUNK)])

    tile_base = (c * NTILE + s) * EPT

    def deg_body(i, _):
        base = tile_base + i * ECHUNK
        pltpu.sync_copy(dst_hbm.at[pl.ds(base, ECHUNK)], dstv)
        pltpu.sync_copy(ew_hbm.at[pl.ds(base, ECHUNK)], ewv)
        pltpu.sync_copy(ewv, deg_sh.at[dstv], add=True)
        return 0
    lax.fori_loop(0, NCHUNK, deg_body, 0)

    plsc.subcore_barrier()
    pltpu.sync_copy(deg_sh.at[pl.ds(s * RPT, RPT)],
                    deg_hbm.at[c, pl.ds(s * RPT, RPT)])


def _sc_gather_deg(emb, xg, dstp, ewp):
    mesh = plsc.VectorSubcoreMesh(core_axis_name="c", subcore_axis_name="s")
    f = pl.kernel(
        _gd_body,
        out_type=(jax.ShapeDtypeStruct((XGP, EMB), jnp.float32),
                  jax.ShapeDtypeStruct((NSC, NP_), jnp.float32)),
        mesh=mesh,
        scratch_types=[
            pltpu.VMEM((ECHUNK,), jnp.int32),
            pltpu.VMEM((ECHUNK, EMB), jnp.float32),
            pltpu.VMEM((ECHUNK,), jnp.int32),
            pltpu.VMEM((ECHUNK,), jnp.float32),
            pltpu.VMEM((RPT,), jnp.float32),
            pltpu.VMEM_SHARED((NP_,), jnp.float32),
            pltpu.SemaphoreType.DMA,
        ],
    )
    return f(emb, xg, dstp, ewp)


def _prep_body(h_ref, m_ref, s_ref, w_ref, o_ref):
    h = h_ref[...] * m_ref[...]
    o_ref[...] = s_ref[...] * jnp.dot(h, w_ref[...],
                                      preferred_element_type=jnp.float32)


def _prep_layer(h, m, s, w):
    return pl.pallas_call(
        _prep_body,
        grid=(NBLK,),
        in_specs=[
            pl.BlockSpec((BLK, HID), lambda i: (i, 0)),
            pl.BlockSpec((BLK, 1), lambda i: (i, 0)),
            pl.BlockSpec((BLK, 1), lambda i: (i, 0)),
            pl.BlockSpec((HID, HID), lambda i: (0, 0)),
        ],
        out_specs=pl.BlockSpec((BLK, HID), lambda i: (i, 0)),
        out_shape=jax.ShapeDtypeStruct((NP_, HID), jnp.float32),
    )(h, m, s, w)


def _combine_body(a0_ref, a1_ref, hs_ref, s_ref, b_ref, w_ref, o_ref):
    t = jax.nn.relu(s_ref[...] * (a0_ref[...] + a1_ref[...] + hs_ref[...])
                    + b_ref[...])
    o_ref[...] = s_ref[...] * jnp.dot(t, w_ref[...],
                                      preferred_element_type=jnp.float32)


def _combine_layer(accp, hs, s, b, w):
    return pl.pallas_call(
        _combine_body,
        grid=(NBLK,),
        in_specs=[
            pl.BlockSpec((None, BLK, HID), lambda i: (0, i, 0)),
            pl.BlockSpec((None, BLK, HID), lambda i: (1, i, 0)),
            pl.BlockSpec((BLK, HID), lambda i: (i, 0)),
            pl.BlockSpec((BLK, 1), lambda i: (i, 0)),
            pl.BlockSpec((1, HID), lambda i: (0, 0)),
            pl.BlockSpec((HID, HID), lambda i: (0, 0)),
        ],
        out_specs=pl.BlockSpec((BLK, HID), lambda i: (i, 0)),
        out_shape=jax.ShapeDtypeStruct((NP_, HID), jnp.float32),
    )(accp, accp, hs, s, b, w)


def _final_body(a0_ref, a1_ref, hs_ref, s_ref, b_ref, batch_ref,
                wm1_ref, bm1_ref, wm2_ref, bm2_ref, o_ref,
                gsum_ref, cnt_ref):
    i = pl.program_id(0)

    @pl.when(i == 0)
    def _():
        gsum_ref[...] = jnp.zeros_like(gsum_ref)
        cnt_ref[...] = jnp.zeros_like(cnt_ref)

    h = jax.nn.relu(s_ref[...] * (a0_ref[...] + a1_ref[...] + hs_ref[...])
                    + b_ref[...])
    seg = batch_ref[...]
    gid = lax.broadcasted_iota(jnp.int32, (G, BLK), 0)
    oh = (seg == gid).astype(jnp.float32)
    gsum_ref[...] += jnp.dot(oh, h, preferred_element_type=jnp.float32)
    cnt_ref[...] += jnp.sum(oh, axis=1, keepdims=True)

    @pl.when(i == NBLK - 1)
    def _():
        g = gsum_ref[...] / jnp.maximum(cnt_ref[...], 1.0)
        a = jax.nn.relu(jnp.dot(g, wm1_ref[...],
                                preferred_element_type=jnp.float32)
                        + bm1_ref[...])
        o = jnp.dot(a, wm2_ref[...],
                    preferred_element_type=jnp.float32) + bm2_ref[...]
        o = o - jnp.max(o, axis=1, keepdims=True)
        eo = jnp.exp(o)
        o_ref[...] = eo / jnp.sum(eo, axis=1, keepdims=True)


def _final_layer(accp, hs, s, b, batch3d, wm1, bm1, wm2, bm2):
    return pl.pallas_call(
        _final_body,
        grid=(NBLK,),
        in_specs=[
            pl.BlockSpec((None, BLK, HID), lambda i: (0, i, 0)),
            pl.BlockSpec((None, BLK, HID), lambda i: (1, i, 0)),
            pl.BlockSpec((BLK, HID), lambda i: (i, 0)),
            pl.BlockSpec((BLK, 1), lambda i: (i, 0)),
            pl.BlockSpec((1, HID), lambda i: (0, 0)),
            pl.BlockSpec((None, 1, BLK), lambda i: (i, 0, 0)),
            pl.BlockSpec((HID, HID // 2), lambda i: (0, 0)),
            pl.BlockSpec((1, HID // 2), lambda i: (0, 0)),
            pl.BlockSpec((HID // 2, 2), lambda i: (0, 0)),
            pl.BlockSpec((1, 2), lambda i: (0, 0)),
        ],
        out_specs=pl.BlockSpec((G, 2), lambda i: (0, 0)),
        out_shape=jax.ShapeDtypeStruct((G, 2), jnp.float32),
        scratch_shapes=[
            pltpu.VMEM((G, HID), jnp.float32),
            pltpu.VMEM((G, 1), jnp.float32),
        ],
    )(accp, accp, hs, s, b, batch3d, wm1, bm1, wm2, bm2)


def kernel(x, edge_index, edge_attr, batch, emb, W1, b1, W2, b2,
           Wm1, bm1, Wm2, bm2):
    src, dst = edge_index[0], edge_index[1]

    spread = (jnp.arange(EP - E, dtype=jnp.int32) * 37) % N
    srcp = jnp.concatenate([src, spread])
    dstp = jnp.concatenate([dst, spread])
    ewp = jnp.pad(edge_attr, (0, EP - E))

    h0 = emb[x]
    deg = jnp.zeros((N,), jnp.float32).at[dst].add(edge_attr) + 1.0

    h0p = jnp.pad(h0, ((0, NP_ - N), (0, 0)))
    xp = jnp.pad(x, (0, NP_ - N)).reshape(NP_, 1)
    degp = jnp.pad(deg, (0, NP_ - N), constant_values=1.0)
    dinv = lax.rsqrt(degp).reshape(NP_, 1)
    mask = (xp != 0).astype(jnp.float32)

    hs1 = _prep_layer(h0p, mask, dinv, W1)

    accp1 = _sc_aggregate(hs1, srcp, dstp, ewp)
    hs2 = _combine_layer(accp1, hs1, dinv, b1.reshape(1, HID), W2)

    accp2 = _sc_aggregate(hs2, srcp, dstp, ewp)

    batchp = jnp.pad(batch, (0, NP_ - N), constant_values=-1)
    batch3d = batchp.reshape(NBLK, 1, BLK)

    return _final_layer(accp2, hs2, dinv, b2.reshape(1, HID), batch3d,
                        Wm1, bm1.reshape(1, HID // 2), Wm2,
                        bm2.reshape(1, 2))


# speedup vs baseline: 8.6905x; 1.1959x over previous
import functools

import jax
import jax.numpy as jnp
from jax import lax
from jax.experimental import pallas as pl
from jax.experimental.pallas import tpu as pltpu
from jax.experimental.pallas import tpu_sc as plsc

N = 10000
E = 320000
G = 128
VOCAB = 100000
EMB = 128
HID = 128

NP_ = 10240
BLK = 256
NBLK = NP_ // BLK

NSC = 2
NTILE = 16
ECHUNK = 128
NCHUNK = 79
EP = NSC * NTILE * NCHUNK * ECHUNK
EPT = NCHUNK * ECHUNK
RPT = NP_ // NTILE


def _agg_body(hs_hbm, src_hbm, dst_hbm, ew_hbm, out_hbm,
              srcv0, srcv1, dstv, ewv0, ewv1, rows0, rows1,
              acc_sh, sem0, sem1):
    c = lax.axis_index("c")
    s = lax.axis_index("s")
    tid = c * NTILE + s
    tb = tid * EPT

    def zero_body(t, _):
        r = t // 8
        j = t % 8
        rows0[r, pl.ds(j * 16, 16)] = jnp.zeros((16,), jnp.float32)
        return 0
    lax.fori_loop(0, ECHUNK * 8, zero_body, 0, unroll=8)
    for k in range(RPT // ECHUNK):
        pltpu.sync_copy(rows0, acc_sh.at[pl.ds(s * RPT + k * ECHUNK, ECHUNK)])
    plsc.subcore_barrier()

    def start_gather(sref, rows, sem, i):
        pltpu.sync_copy(src_hbm.at[pl.ds(tb + i * ECHUNK, ECHUNK)], sref)
        pltpu.async_copy(hs_hbm.at[sref], rows, sem)

    def scale_and_scatter(rows, ewv, i):
        pltpu.sync_copy(ew_hbm.at[pl.ds(tb + i * ECHUNK, ECHUNK)], ewv)
        pltpu.sync_copy(dst_hbm.at[pl.ds(tb + i * ECHUNK, ECHUNK)], dstv)

        def scale_body(g, _):
            ewg = ewv[pl.ds(g * 16, 16)]
            for l in range(16):
                spl = jnp.full((16,), ewg[l], jnp.float32)
                for j in range(8):
                    rows[g * 16 + l, pl.ds(j * 16, 16)] = \
                        rows[g * 16 + l, pl.ds(j * 16, 16)] * spl
            return 0
        lax.fori_loop(0, ECHUNK // 16, scale_body, 0)
        pltpu.sync_copy(rows, acc_sh.at[dstv], add=True)

    start_gather(srcv0, rows0, sem0, 0)

    def pair_body(p, _):
        i0 = p * 2
        start_gather(srcv1, rows1, sem1, i0 + 1)
        pltpu.make_async_copy(hs_hbm.at[srcv0], rows0, sem0).wait()
        scale_and_scatter(rows0, ewv0, i0)
        start_gather(srcv0, rows0, sem0, i0 + 2)
        pltpu.make_async_copy(hs_hbm.at[srcv1], rows1, sem1).wait()
        scale_and_scatter(rows1, ewv1, i0 + 1)
        return 0
    lax.fori_loop(0, (NCHUNK - 1) // 2, pair_body, 0)

    pltpu.make_async_copy(hs_hbm.at[srcv0], rows0, sem0).wait()
    scale_and_scatter(rows0, ewv0, NCHUNK - 1)

    plsc.subcore_barrier()
    pltpu.sync_copy(acc_sh.at[pl.ds(s * RPT, RPT)],
                    out_hbm.at[c, pl.ds(s * RPT, RPT)])


def _sc_aggregate(hs, srcp, dstp, ewp):
    mesh = plsc.VectorSubcoreMesh(core_axis_name="c", subcore_axis_name="s")
    f = pl.kernel(
        _agg_body,
        out_type=jax.ShapeDtypeStruct((NSC, NP_, HID), jnp.float32),
        mesh=mesh,
        scratch_types=[
            pltpu.VMEM((ECHUNK,), jnp.int32),
            pltpu.VMEM((ECHUNK,), jnp.int32),
            pltpu.VMEM((ECHUNK,), jnp.int32),
            pltpu.VMEM((ECHUNK,), jnp.float32),
            pltpu.VMEM((ECHUNK,), jnp.float32),
            pltpu.VMEM((ECHUNK, HID), jnp.float32),
            pltpu.VMEM((ECHUNK, HID), jnp.float32),
            pltpu.VMEM_SHARED((NP_, HID), jnp.float32),
            pltpu.SemaphoreType.DMA,
            pltpu.SemaphoreType.DMA,
        ],
    )
    return f(hs, srcp, dstp, ewp)


XGP = NSC * NTILE * 3 * ECHUNK


def _gd_body(emb_hbm, xg_hbm, dst_hbm, ew_hbm, h0_hbm, deg_hbm,
             idxv, rows_v, dstv, ewv, zv, deg_sh, sem):
    c = lax.axis_index("c")
    s = lax.axis_index("s")
    wid = c * NTILE + s

    def zero_body(t, _):
        zv[pl.ds(t * 16, 16)] = jnp.zeros((16,), jnp.float32)
        return 0
    lax.fori_loop(0, RPT // 16, zero_body, 0, unroll=8)
    pltpu.sync_copy(zv, deg_sh.at[pl.ds(s * RPT, RPT)])
    plsc.subcore_barrier()

    for k in range(3):
        base = (wid * 3 + k) * ECHUNK
        pltpu.sync_copy(xg_hbm.at[pl.ds(base, ECHUNK)], idxv)
        pltpu.async_copy(emb_hbm.at[idxv], rows_v, sem).wait()
        pltpu.sync_copy(rows_v, h0_hbm.at[pl.ds(base, ECHUNK)])

    tile_base = (c * NTILE + s) * EPT

    def deg_body(i, _):
        base = tile_base + i * ECHUNK
        pltpu.sync_copy(dst_hbm.at[pl.ds(base, ECHUNK)], dstv)
        pltpu.sync_copy(ew_hbm.at[pl.ds(base, ECHUNK)], ewv)
        pltpu.sync_copy(ewv, deg_sh.at[dstv], add=True)
        return 0
    lax.fori_loop(0, NCHUNK, deg_body, 0)

    plsc.subcore_barrier()
    pltpu.sync_copy(deg_sh.at[pl.ds(s * RPT, RPT)],
                    deg_hbm.at[c, pl.ds(s * RPT, RPT)])


def _sc_gather_deg(emb, xg, dstp, ewp):
    mesh = plsc.VectorSubcoreMesh(core_axis_name="c", subcore_axis_name="s")
    f = pl.kernel(
        _gd_body,
        out_type=(jax.ShapeDtypeStruct((XGP, EMB), jnp.float32),
                  jax.ShapeDtypeStruct((NSC, NP_), jnp.float32)),
        mesh=mesh,
        scratch_types=[
            pltpu.VMEM((ECHUNK,), jnp.int32),
            pltpu.VMEM((ECHUNK, EMB), jnp.float32),
            pltpu.VMEM((ECHUNK,), jnp.int32),
            pltpu.VMEM((ECHUNK,), jnp.float32),
            pltpu.VMEM((RPT,), jnp.float32),
            pltpu.VMEM_SHARED((NP_,), jnp.float32),
            pltpu.SemaphoreType.DMA,
        ],
    )
    return f(emb, xg, dstp, ewp)


def _prep_body(h_ref, m_ref, s_ref, w_ref, o_ref):
    h = h_ref[...] * m_ref[...]
    o_ref[...] = s_ref[...] * jnp.dot(h, w_ref[...],
                                      preferred_element_type=jnp.float32)


def _prep_layer(h, m, s, w):
    return pl.pallas_call(
        _prep_body,
        grid=(NBLK,),
        in_specs=[
            pl.BlockSpec((BLK, HID), lambda i: (i, 0)),
            pl.BlockSpec((BLK, 1), lambda i: (i, 0)),
            pl.BlockSpec((BLK, 1), lambda i: (i, 0)),
            pl.BlockSpec((HID, HID), lambda i: (0, 0)),
        ],
        out_specs=pl.BlockSpec((BLK, HID), lambda i: (i, 0)),
        out_shape=jax.ShapeDtypeStruct((NP_, HID), jnp.float32),
    )(h, m, s, w)


def _combine_body(a0_ref, a1_ref, hs_ref, s_ref, b_ref, w_ref, o_ref):
    t = jax.nn.relu(s_ref[...] * (a0_ref[...] + a1_ref[...] + hs_ref[...])
                    + b_ref[...])
    o_ref[...] = s_ref[...] * jnp.dot(t, w_ref[...],
                                      preferred_element_type=jnp.float32)


def _combine_layer(accp, hs, s, b, w):
    return pl.pallas_call(
        _combine_body,
        grid=(NBLK,),
        in_specs=[
            pl.BlockSpec((None, BLK, HID), lambda i: (0, i, 0)),
            pl.BlockSpec((None, BLK, HID), lambda i: (1, i, 0)),
            pl.BlockSpec((BLK, HID), lambda i: (i, 0)),
            pl.BlockSpec((BLK, 1), lambda i: (i, 0)),
            pl.BlockSpec((1, HID), lambda i: (0, 0)),
            pl.BlockSpec((HID, HID), lambda i: (0, 0)),
        ],
        out_specs=pl.BlockSpec((BLK, HID), lambda i: (i, 0)),
        out_shape=jax.ShapeDtypeStruct((NP_, HID), jnp.float32),
    )(accp, accp, hs, s, b, w)


def _final_body(a0_ref, a1_ref, hs_ref, s_ref, b_ref, batch_ref,
                wm1_ref, bm1_ref, wm2_ref, bm2_ref, o_ref,
                gsum_ref, cnt_ref):
    i = pl.program_id(0)

    @pl.when(i == 0)
    def _():
        gsum_ref[...] = jnp.zeros_like(gsum_ref)
        cnt_ref[...] = jnp.zeros_like(cnt_ref)

    h = jax.nn.relu(s_ref[...] * (a0_ref[...] + a1_ref[...] + hs_ref[...])
                    + b_ref[...])
    seg = batch_ref[...]
    gid = lax.broadcasted_iota(jnp.int32, (G, BLK), 0)
    oh = (seg == gid).astype(jnp.float32)
    gsum_ref[...] += jnp.dot(oh, h, preferred_element_type=jnp.float32)
    cnt_ref[...] += jnp.sum(oh, axis=1, keepdims=True)

    @pl.when(i == NBLK - 1)
    def _():
        g = gsum_ref[...] / jnp.maximum(cnt_ref[...], 1.0)
        a = jax.nn.relu(jnp.dot(g, wm1_ref[...],
                                preferred_element_type=jnp.float32)
                        + bm1_ref[...])
        o = jnp.dot(a, wm2_ref[...],
                    preferred_element_type=jnp.float32) + bm2_ref[...]
        o = o - jnp.max(o, axis=1, keepdims=True)
        eo = jnp.exp(o)
        o_ref[...] = eo / jnp.sum(eo, axis=1, keepdims=True)


def _final_layer(accp, hs, s, b, batch3d, wm1, bm1, wm2, bm2):
    return pl.pallas_call(
        _final_body,
        grid=(NBLK,),
        in_specs=[
            pl.BlockSpec((None, BLK, HID), lambda i: (0, i, 0)),
            pl.BlockSpec((None, BLK, HID), lambda i: (1, i, 0)),
            pl.BlockSpec((BLK, HID), lambda i: (i, 0)),
            pl.BlockSpec((BLK, 1), lambda i: (i, 0)),
            pl.BlockSpec((1, HID), lambda i: (0, 0)),
            pl.BlockSpec((None, 1, BLK), lambda i: (i, 0, 0)),
            pl.BlockSpec((HID, HID // 2), lambda i: (0, 0)),
            pl.BlockSpec((1, HID // 2), lambda i: (0, 0)),
            pl.BlockSpec((HID // 2, 2), lambda i: (0, 0)),
            pl.BlockSpec((1, 2), lambda i: (0, 0)),
        ],
        out_specs=pl.BlockSpec((G, 2), lambda i: (0, 0)),
        out_shape=jax.ShapeDtypeStruct((G, 2), jnp.float32),
        scratch_shapes=[
            pltpu.VMEM((G, HID), jnp.float32),
            pltpu.VMEM((G, 1), jnp.float32),
        ],
    )(accp, accp, hs, s, b, batch3d, wm1, bm1, wm2, bm2)


def kernel(x, edge_index, edge_attr, batch, emb, W1, b1, W2, b2,
           Wm1, bm1, Wm2, bm2):
    src, dst = edge_index[0], edge_index[1]

    spread = (jnp.arange(EP - E, dtype=jnp.int32) * 37) % N
    srcp = jnp.concatenate([src, spread])
    dstp = jnp.concatenate([dst, spread])
    ewp = jnp.pad(edge_attr, (0, EP - E))

    h0 = emb[x]
    deg = jnp.zeros((N,), jnp.float32).at[dst].add(edge_attr) + 1.0

    h0p = jnp.pad(h0, ((0, NP_ - N), (0, 0)))
    xp = jnp.pad(x, (0, NP_ - N)).reshape(NP_, 1)
    degp = jnp.pad(deg, (0, NP_ - N), constant_values=1.0)
    dinv = lax.rsqrt(degp).reshape(NP_, 1)
    mask = (xp != 0).astype(jnp.float32)

    hs1 = _prep_layer(h0p, mask, dinv, W1)

    accp1 = _sc_aggregate(hs1, srcp, dstp, ewp)
    hs2 = _combine_layer(accp1, hs1, dinv, b1.reshape(1, HID), W2)

    accp2 = _sc_aggregate(hs2, srcp, dstp, ewp)

    batchp = jnp.pad(batch, (0, NP_ - N), constant_values=-1)
    batch3d = batchp.reshape(NBLK, 1, BLK)

    return _final_layer(accp2, hs2, dinv, b2.reshape(1, HID), batch3d,
                        Wm1, bm1.reshape(1, HID // 2), Wm2,
                        bm2.reshape(1, 2))


# speedup vs baseline: 13.4108x; 1.5432x over previous
import functools

import jax
import jax.numpy as jnp
from jax import lax
from jax.experimental import pallas as pl
from jax.experimental.pallas import tpu as pltpu
from jax.experimental.pallas import tpu_sc as plsc

N = 10000
E = 320000
G = 128
VOCAB = 100000
EMB = 128
HID = 128

NP_ = 10240
BLK = 256
NBLK = NP_ // BLK

NSC = 2
NTILE = 16
ECHUNK = 128
NCHUNK = 80
EP = NSC * NTILE * NCHUNK * ECHUNK
EPT = NCHUNK * ECHUNK
RPT = NP_ // NTILE


def _agg_body(hs_hbm, src_hbm, dst_hbm, ew_hbm, out_hbm,
              srcv0, srcv1, dstv, ewv0, ewv1, rows0, rows1,
              acc_sh, sem0, sem1):
    c = lax.axis_index("c")
    s = lax.axis_index("s")
    tid = c * NTILE + s
    tb = tid * EPT

    def zero_body(t, _):
        r = t // 8
        j = t % 8
        rows0[r, pl.ds(j * 16, 16)] = jnp.zeros((16,), jnp.float32)
        return 0
    lax.fori_loop(0, ECHUNK * 8, zero_body, 0, unroll=8)
    for k in range(RPT // ECHUNK):
        pltpu.sync_copy(rows0, acc_sh.at[pl.ds(s * RPT + k * ECHUNK, ECHUNK)])
    plsc.subcore_barrier()

    def start_gather(sref, rows, sem, i):
        pltpu.sync_copy(src_hbm.at[pl.ds(tb + i * ECHUNK, ECHUNK)], sref)
        pltpu.async_copy(hs_hbm.at[sref], rows, sem)

    def scale_and_scatter(rows, ewv, i):
        pltpu.sync_copy(ew_hbm.at[pl.ds(tb + i * ECHUNK, ECHUNK)], ewv)
        pltpu.sync_copy(dst_hbm.at[pl.ds(tb + i * ECHUNK, ECHUNK)], dstv)

        def scale_body(g, _):
            ewg = ewv[pl.ds(g * 16, 16)]
            for l in range(16):
                spl = jnp.full((16,), ewg[l], jnp.float32)
                for j in range(8):
                    rows[g * 16 + l, pl.ds(j * 16, 16)] = \
                        rows[g * 16 + l, pl.ds(j * 16, 16)] * spl
            return 0
        lax.fori_loop(0, ECHUNK // 16, scale_body, 0)
        pltpu.sync_copy(rows, acc_sh.at[dstv], add=True)

    start_gather(srcv0, rows0, sem0, 0)

    def pair_body(p, _):
        i0 = p * 2
        start_gather(srcv1, rows1, sem1, i0 + 1)
        pltpu.make_async_copy(hs_hbm.at[srcv0], rows0, sem0).wait()
        scale_and_scatter(rows0, ewv0, i0)
        start_gather(srcv0, rows0, sem0, i0 + 2)
        pltpu.make_async_copy(hs_hbm.at[srcv1], rows1, sem1).wait()
        scale_and_scatter(rows1, ewv1, i0 + 1)
        return 0
    lax.fori_loop(0, NCHUNK // 2 - 1, pair_body, 0)

    start_gather(srcv1, rows1, sem1, NCHUNK - 1)
    pltpu.make_async_copy(hs_hbm.at[srcv0], rows0, sem0).wait()
    scale_and_scatter(rows0, ewv0, NCHUNK - 2)
    pltpu.make_async_copy(hs_hbm.at[srcv1], rows1, sem1).wait()
    scale_and_scatter(rows1, ewv1, NCHUNK - 1)

    plsc.subcore_barrier()
    pltpu.sync_copy(acc_sh.at[pl.ds(s * RPT, RPT)],
                    out_hbm.at[c, pl.ds(s * RPT, RPT)])


def _sc_aggregate(hs, srcp, dstp, ewp):
    mesh = plsc.VectorSubcoreMesh(core_axis_name="c", subcore_axis_name="s")
    f = pl.kernel(
        _agg_body,
        out_type=jax.ShapeDtypeStruct((NSC, NP_, HID), jnp.float32),
        mesh=mesh,
        scratch_types=[
            pltpu.VMEM((ECHUNK,), jnp.int32),
            pltpu.VMEM((ECHUNK,), jnp.int32),
            pltpu.VMEM((ECHUNK,), jnp.int32),
            pltpu.VMEM((ECHUNK,), jnp.float32),
            pltpu.VMEM((ECHUNK,), jnp.float32),
            pltpu.VMEM((ECHUNK, HID), jnp.float32),
            pltpu.VMEM((ECHUNK, HID), jnp.float32),
            pltpu.VMEM_SHARED((NP_, HID), jnp.float32),
            pltpu.SemaphoreType.DMA,
            pltpu.SemaphoreType.DMA,
        ],
    )
    return f(hs, srcp, dstp, ewp)


XGP = NSC * NTILE * 3 * ECHUNK


def _gd_body(emb_hbm, xg_hbm, dst2_hbm, ew2_hbm, h0_hbm, deg_hbm,
             idxv, rows_v, dst_t, ew_t, zv, deg_sh, sem):
    c = lax.axis_index("c")
    s = lax.axis_index("s")
    wid = c * NTILE + s

    pltpu.sync_copy(dst2_hbm.at[pl.ds(wid * NCHUNK, NCHUNK)], dst_t)
    pltpu.sync_copy(ew2_hbm.at[pl.ds(wid * NCHUNK, NCHUNK)], ew_t)

    def zero_body(t, _):
        zv[pl.ds(t * 16, 16)] = jnp.zeros((16,), jnp.float32)
        return 0
    lax.fori_loop(0, RPT // 16, zero_body, 0, unroll=8)
    pltpu.sync_copy(zv, deg_sh.at[pl.ds(s * RPT, RPT)])
    plsc.subcore_barrier()

    for k in range(3):
        base = (wid * 3 + k) * ECHUNK
        pltpu.sync_copy(xg_hbm.at[pl.ds(base, ECHUNK)], idxv)
        pltpu.async_copy(emb_hbm.at[idxv], rows_v, sem).wait()
        pltpu.sync_copy(rows_v, h0_hbm.at[pl.ds(base, ECHUNK)])

    def deg_body(i, _):
        pltpu.sync_copy(ew_t.at[i], deg_sh.at[dst_t.at[i]], add=True)
        return 0
    lax.fori_loop(0, NCHUNK, deg_body, 0)

    plsc.subcore_barrier()
    pltpu.sync_copy(deg_sh.at[pl.ds(s * RPT, RPT)],
                    deg_hbm.at[c, pl.ds(s * RPT, RPT)])


def _sc_gather_deg(emb, xg, dst2, ew2):
    mesh = plsc.VectorSubcoreMesh(core_axis_name="c", subcore_axis_name="s")
    f = pl.kernel(
        _gd_body,
        out_type=(jax.ShapeDtypeStruct((XGP, EMB), jnp.float32),
                  jax.ShapeDtypeStruct((NSC, NP_), jnp.float32)),
        mesh=mesh,
        scratch_types=[
            pltpu.VMEM((ECHUNK,), jnp.int32),
            pltpu.VMEM((ECHUNK, EMB), jnp.float32),
            pltpu.VMEM((NCHUNK, ECHUNK), jnp.int32),
            pltpu.VMEM((NCHUNK, ECHUNK), jnp.float32),
            pltpu.VMEM((RPT,), jnp.float32),
            pltpu.VMEM_SHARED((NP_,), jnp.float32),
            pltpu.SemaphoreType.DMA,
        ],
    )
    return f(emb, xg, dst2, ew2)


def _prep_body(h_ref, m_ref, d0_ref, d1_ref, w_ref, o_ref, s_ref):
    dv = lax.rsqrt(d0_ref[...] + d1_ref[...] + 1.0)
    s_ref[...] = dv
    h = h_ref[...] * m_ref[...]
    o_ref[...] = dv * jnp.dot(h, w_ref[...],
                              preferred_element_type=jnp.float32)


def _prep_layer(h, m, degp3, w):
    return pl.pallas_call(
        _prep_body,
        grid=(NBLK,),
        in_specs=[
            pl.BlockSpec((BLK, HID), lambda i: (i, 0)),
            pl.BlockSpec((BLK, 1), lambda i: (i, 0)),
            pl.BlockSpec((None, BLK, 1), lambda i: (0, i, 0)),
            pl.BlockSpec((None, BLK, 1), lambda i: (1, i, 0)),
            pl.BlockSpec((HID, HID), lambda i: (0, 0)),
        ],
        out_specs=[
            pl.BlockSpec((BLK, HID), lambda i: (i, 0)),
            pl.BlockSpec((BLK, 1), lambda i: (i, 0)),
        ],
        out_shape=[
            jax.ShapeDtypeStruct((NP_, HID), jnp.float32),
            jax.ShapeDtypeStruct((NP_, 1), jnp.float32),
        ],
    )(h, m, degp3, degp3, w)


def _combine_body(a0_ref, a1_ref, hs_ref, s_ref, b_ref, w_ref, o_ref):
    t = jax.nn.relu(s_ref[...] * (a0_ref[...] + a1_ref[...] + hs_ref[...])
                    + b_ref[...])
    o_ref[...] = s_ref[...] * jnp.dot(t, w_ref[...],
                                      preferred_element_type=jnp.float32)


def _combine_layer(accp, hs, s, b, w):
    return pl.pallas_call(
        _combine_body,
        grid=(NBLK,),
        in_specs=[
            pl.BlockSpec((None, BLK, HID), lambda i: (0, i, 0)),
            pl.BlockSpec((None, BLK, HID), lambda i: (1, i, 0)),
            pl.BlockSpec((BLK, HID), lambda i: (i, 0)),
            pl.BlockSpec((BLK, 1), lambda i: (i, 0)),
            pl.BlockSpec((1, HID), lambda i: (0, 0)),
            pl.BlockSpec((HID, HID), lambda i: (0, 0)),
        ],
        out_specs=pl.BlockSpec((BLK, HID), lambda i: (i, 0)),
        out_shape=jax.ShapeDtypeStruct((NP_, HID), jnp.float32),
    )(accp, accp, hs, s, b, w)


def _final_body(a0_ref, a1_ref, hs_ref, s_ref, b_ref, batch_ref,
                wm1_ref, bm1_ref, wm2_ref, bm2_ref, o_ref,
                gsum_ref, cnt_ref):
    i = pl.program_id(0)

    @pl.when(i == 0)
    def _():
        gsum_ref[...] = jnp.zeros_like(gsum_ref)
        cnt_ref[...] = jnp.zeros_like(cnt_ref)

    h = jax.nn.relu(s_ref[...] * (a0_ref[...] + a1_ref[...] + hs_ref[...])
                    + b_ref[...])
    seg = batch_ref[...]
    gid = lax.broadcasted_iota(jnp.int32, (G, BLK), 0)
    oh = (seg == gid).astype(jnp.float32)
    gsum_ref[...] += jnp.dot(oh, h, preferred_element_type=jnp.float32)
    cnt_ref[...] += jnp.sum(oh, axis=1, keepdims=True)

    @pl.when(i == NBLK - 1)
    def _():
        g = gsum_ref[...] / jnp.maximum(cnt_ref[...], 1.0)
        a = jax.nn.relu(jnp.dot(g, wm1_ref[...],
                                preferred_element_type=jnp.float32)
                        + bm1_ref[...])
        o = jnp.dot(a, wm2_ref[...],
                    preferred_element_type=jnp.float32) + bm2_ref[...]
        o = o - jnp.max(o, axis=1, keepdims=True)
        eo = jnp.exp(o)
        o_ref[...] = eo / jnp.sum(eo, axis=1, keepdims=True)


def _final_layer(accp, hs, s, b, batch3d, wm1, bm1, wm2, bm2):
    return pl.pallas_call(
        _final_body,
        grid=(NBLK,),
        in_specs=[
            pl.BlockSpec((None, BLK, HID), lambda i: (0, i, 0)),
            pl.BlockSpec((None, BLK, HID), lambda i: (1, i, 0)),
            pl.BlockSpec((BLK, HID), lambda i: (i, 0)),
            pl.BlockSpec((BLK, 1), lambda i: (i, 0)),
            pl.BlockSpec((1, HID), lambda i: (0, 0)),
            pl.BlockSpec((None, 1, BLK), lambda i: (i, 0, 0)),
            pl.BlockSpec((HID, HID // 2), lambda i: (0, 0)),
            pl.BlockSpec((1, HID // 2), lambda i: (0, 0)),
            pl.BlockSpec((HID // 2, 2), lambda i: (0, 0)),
            pl.BlockSpec((1, 2), lambda i: (0, 0)),
        ],
        out_specs=pl.BlockSpec((G, 2), lambda i: (0, 0)),
        out_shape=jax.ShapeDtypeStruct((G, 2), jnp.float32),
        scratch_shapes=[
            pltpu.VMEM((G, HID), jnp.float32),
            pltpu.VMEM((G, 1), jnp.float32),
        ],
    )(accp, accp, hs, s, b, batch3d, wm1, bm1, wm2, bm2)


def kernel(x, edge_index, edge_attr, batch, emb, W1, b1, W2, b2,
           Wm1, bm1, Wm2, bm2):
    src, dst = edge_index[0], edge_index[1]

    spread = (jnp.arange(EP - E, dtype=jnp.int32) * 37) % N
    srcp = jnp.concatenate([src, spread])
    dstp = jnp.concatenate([dst, spread])
    ewp = jnp.pad(edge_attr, (0, EP - E))

    xg = jnp.concatenate(
        [x, (jnp.arange(XGP - N, dtype=jnp.int32) * 37 + 11) % VOCAB])
    h0f, degp = _sc_gather_deg(emb, xg,
                               dstp.reshape(EP // ECHUNK, ECHUNK),
                               ewp.reshape(EP // ECHUNK, ECHUNK))
    h0p = h0f[:NP_]
    xp = jnp.pad(x, (0, NP_ - N)).reshape(NP_, 1)
    mask = (xp != 0).astype(jnp.float32)

    hs1, dinv = _prep_layer(h0p, mask, degp.reshape(NSC, NP_, 1), W1)

    accp1 = _sc_aggregate(hs1, srcp, dstp, ewp)
    hs2 = _combine_layer(accp1, hs1, dinv, b1.reshape(1, HID), W2)

    accp2 = _sc_aggregate(hs2, srcp, dstp, ewp)

    batchp = jnp.pad(batch, (0, NP_ - N), constant_values=-1)
    batch3d = batchp.reshape(NBLK, 1, BLK)

    return _final_layer(accp2, hs2, dinv, b2.reshape(1, HID), batch3d,
                        Wm1, bm1.reshape(1, HID // 2), Wm2,
                        bm2.reshape(1, 2))


# speedup vs baseline: 14.2997x; 1.0663x over previous
import functools

import jax
import jax.numpy as jnp
from jax import lax
from jax.experimental import pallas as pl
from jax.experimental.pallas import tpu as pltpu
from jax.experimental.pallas import tpu_sc as plsc

N = 10000
E = 320000
G = 128
VOCAB = 100000
EMB = 128
HID = 128

NP_ = 10240
BLK = 256
NBLK = NP_ // BLK

NSC = 2
NTILE = 16
ECHUNK = 128
NCHUNK = 80
EP = NSC * NTILE * NCHUNK * ECHUNK
EPT = NCHUNK * ECHUNK
RPT = NP_ // NTILE


def _agg_body(hs_hbm, src_hbm, de_hbm, ew_hbm, out_hbm,
              srcv0, srcv1, dstv0, dstv1, ewv0, ewv1, rows0, rows1,
              acc_sh, sem0, sem1, sc0, sc1):
    c = lax.axis_index("c")
    s = lax.axis_index("s")
    tid = c * NTILE + s
    tb = tid * EPT

    def zero_body(t, _):
        r = t // 8
        j = t % 8
        rows0[r, pl.ds(j * 16, 16)] = jnp.zeros((16,), jnp.float32)
        return 0
    lax.fori_loop(0, ECHUNK * 8, zero_body, 0, unroll=8)
    for k in range(RPT // ECHUNK):
        pltpu.sync_copy(rows0, acc_sh.at[pl.ds(s * RPT + k * ECHUNK, ECHUNK)])
    plsc.subcore_barrier()

    def start_gather(sref, rows, sem, i):
        pltpu.sync_copy(src_hbm.at[pl.ds(tb + i * ECHUNK, ECHUNK)], sref)
        pltpu.async_copy(hs_hbm.at[sref], rows, sem)

    def scale_and_scatter(rows, ewv, dstv, sc, i):
        pltpu.sync_copy(de_hbm.at[pl.ds(tb + i * ECHUNK, ECHUNK)], dstv)
        pltpu.sync_copy(ew_hbm.at[pl.ds(tb + i * ECHUNK, ECHUNK)], ewv)

        def scale_body(g, _):
            ewg = ewv[pl.ds(g * 16, 16)]
            for l in range(16):
                spl = jnp.full((16,), ewg[l], jnp.float32)
                for j in range(8):
                    rows[g * 16 + l, pl.ds(j * 16, 16)] = \
                        rows[g * 16 + l, pl.ds(j * 16, 16)] * spl
            return 0
        lax.fori_loop(0, ECHUNK // 16, scale_body, 0)
        pltpu.async_copy(rows, acc_sh.at[dstv], sc, add=True)

    def wait_gather(sref, rows, sem):
        pltpu.make_async_copy(hs_hbm.at[sref], rows, sem).wait()

    def wait_scatter(rows, dstv, sc):
        pltpu.make_async_copy(rows, acc_sh.at[dstv], sc).wait()

    start_gather(srcv0, rows0, sem0, 0)
    start_gather(srcv1, rows1, sem1, 1)

    def pair_body(p, _):
        i0 = p * 2
        wait_gather(srcv0, rows0, sem0)
        scale_and_scatter(rows0, ewv0, dstv0, sc0, i0)
        wait_gather(srcv1, rows1, sem1)
        scale_and_scatter(rows1, ewv1, dstv1, sc1, i0 + 1)
        wait_scatter(rows0, dstv0, sc0)
        start_gather(srcv0, rows0, sem0, i0 + 2)
        wait_scatter(rows1, dstv1, sc1)
        start_gather(srcv1, rows1, sem1, i0 + 3)
        return 0
    lax.fori_loop(0, NCHUNK // 2 - 1, pair_body, 0)

    wait_gather(srcv0, rows0, sem0)
    scale_and_scatter(rows0, ewv0, dstv0, sc0, NCHUNK - 2)
    wait_gather(srcv1, rows1, sem1)
    scale_and_scatter(rows1, ewv1, dstv1, sc1, NCHUNK - 1)
    wait_scatter(rows0, dstv0, sc0)
    wait_scatter(rows1, dstv1, sc1)

    plsc.subcore_barrier()
    pltpu.sync_copy(acc_sh.at[pl.ds(s * RPT, RPT)],
                    out_hbm.at[c, pl.ds(s * RPT, RPT)])


def _sc_aggregate(hs, srcp, dstp, ewp):
    mesh = plsc.VectorSubcoreMesh(core_axis_name="c", subcore_axis_name="s")
    f = pl.kernel(
        _agg_body,
        out_type=jax.ShapeDtypeStruct((NSC, NP_, HID), jnp.float32),
        mesh=mesh,
        scratch_types=[
            pltpu.VMEM((ECHUNK,), jnp.int32),
            pltpu.VMEM((ECHUNK,), jnp.int32),
            pltpu.VMEM((ECHUNK,), jnp.int32),
            pltpu.VMEM((ECHUNK,), jnp.int32),
            pltpu.VMEM((ECHUNK,), jnp.float32),
            pltpu.VMEM((ECHUNK,), jnp.float32),
            pltpu.VMEM((ECHUNK, HID), jnp.float32),
            pltpu.VMEM((ECHUNK, HID), jnp.float32),
            pltpu.VMEM_SHARED((NP_, HID), jnp.float32),
            pltpu.SemaphoreType.DMA,
            pltpu.SemaphoreType.DMA,
            pltpu.SemaphoreType.DMA,
            pltpu.SemaphoreType.DMA,
        ],
    )
    return f(hs, srcp, dstp, ewp)


XGP = NSC * NTILE * 3 * ECHUNK


def _gd_body(emb_hbm, xg_hbm, dst2_hbm, ew2_hbm, h0_hbm, deg_hbm,
             idxv, rows_v, dst_t, ew_t, zv, deg_sh, sem):
    c = lax.axis_index("c")
    s = lax.axis_index("s")
    wid = c * NTILE + s

    pltpu.sync_copy(dst2_hbm.at[pl.ds(wid * NCHUNK, NCHUNK)], dst_t)
    pltpu.sync_copy(ew2_hbm.at[pl.ds(wid * NCHUNK, NCHUNK)], ew_t)

    def zero_body(t, _):
        zv[pl.ds(t * 16, 16)] = jnp.zeros((16,), jnp.float32)
        return 0
    lax.fori_loop(0, RPT // 16, zero_body, 0, unroll=8)
    pltpu.sync_copy(zv, deg_sh.at[pl.ds(s * RPT, RPT)])
    plsc.subcore_barrier()

    for k in range(3):
        base = (wid * 3 + k) * ECHUNK
        pltpu.sync_copy(xg_hbm.at[pl.ds(base, ECHUNK)], idxv)
        pltpu.async_copy(emb_hbm.at[idxv], rows_v, sem).wait()
        pltpu.sync_copy(rows_v, h0_hbm.at[pl.ds(base, ECHUNK)])

    def deg_body(i, _):
        pltpu.sync_copy(ew_t.at[i], deg_sh.at[dst_t.at[i]], add=True)
        return 0
    lax.fori_loop(0, NCHUNK, deg_body, 0)

    plsc.subcore_barrier()
    pltpu.sync_copy(deg_sh.at[pl.ds(s * RPT, RPT)],
                    deg_hbm.at[c, pl.ds(s * RPT, RPT)])


def _sc_gather_deg(emb, xg, dst2, ew2):
    mesh = plsc.VectorSubcoreMesh(core_axis_name="c", subcore_axis_name="s")
    f = pl.kernel(
        _gd_body,
        out_type=(jax.ShapeDtypeStruct((XGP, EMB), jnp.float32),
                  jax.ShapeDtypeStruct((NSC, NP_), jnp.float32)),
        mesh=mesh,
        scratch_types=[
            pltpu.VMEM((ECHUNK,), jnp.int32),
            pltpu.VMEM((ECHUNK, EMB), jnp.float32),
            pltpu.VMEM((NCHUNK, ECHUNK), jnp.int32),
            pltpu.VMEM((NCHUNK, ECHUNK), jnp.float32),
            pltpu.VMEM((RPT,), jnp.float32),
            pltpu.VMEM_SHARED((NP_,), jnp.float32),
            pltpu.SemaphoreType.DMA,
        ],
    )
    return f(emb, xg, dst2, ew2)


def _prep_body(h_ref, m_ref, d0_ref, d1_ref, w_ref, o_ref, s_ref):
    dv = lax.rsqrt(d0_ref[...] + d1_ref[...] + 1.0)
    s_ref[...] = dv
    h = h_ref[...] * m_ref[...]
    o_ref[...] = dv * jnp.dot(h, w_ref[...],
                              preferred_element_type=jnp.float32)


def _prep_layer(h, m, degp3, w):
    return pl.pallas_call(
        _prep_body,
        grid=(NBLK,),
        in_specs=[
            pl.BlockSpec((BLK, HID), lambda i: (i, 0)),
            pl.BlockSpec((BLK, 1), lambda i: (i, 0)),
            pl.BlockSpec((None, BLK, 1), lambda i: (0, i, 0)),
            pl.BlockSpec((None, BLK, 1), lambda i: (1, i, 0)),
            pl.BlockSpec((HID, HID), lambda i: (0, 0)),
        ],
        out_specs=[
            pl.BlockSpec((BLK, HID), lambda i: (i, 0)),
            pl.BlockSpec((BLK, 1), lambda i: (i, 0)),
        ],
        out_shape=[
            jax.ShapeDtypeStruct((NP_, HID), jnp.float32),
            jax.ShapeDtypeStruct((NP_, 1), jnp.float32),
        ],
    )(h, m, degp3, degp3, w)


def _combine_body(a0_ref, a1_ref, hs_ref, s_ref, b_ref, w_ref, o_ref):
    t = jax.nn.relu(s_ref[...] * (a0_ref[...] + a1_ref[...] + hs_ref[...])
                    + b_ref[...])
    o_ref[...] = s_ref[...] * jnp.dot(t, w_ref[...],
                                      preferred_element_type=jnp.float32)


def _combine_layer(accp, hs, s, b, w):
    return pl.pallas_call(
        _combine_body,
        grid=(NBLK,),
        in_specs=[
            pl.BlockSpec((None, BLK, HID), lambda i: (0, i, 0)),
            pl.BlockSpec((None, BLK, HID), lambda i: (1, i, 0)),
            pl.BlockSpec((BLK, HID), lambda i: (i, 0)),
            pl.BlockSpec((BLK, 1), lambda i: (i, 0)),
            pl.BlockSpec((1, HID), lambda i: (0, 0)),
            pl.BlockSpec((HID, HID), lambda i: (0, 0)),
        ],
        out_specs=pl.BlockSpec((BLK, HID), lambda i: (i, 0)),
        out_shape=jax.ShapeDtypeStruct((NP_, HID), jnp.float32),
    )(accp, accp, hs, s, b, w)


def _final_body(a0_ref, a1_ref, hs_ref, s_ref, b_ref, batch_ref,
                wm1_ref, bm1_ref, wm2_ref, bm2_ref, o_ref,
                gsum_ref, cnt_ref):
    i = pl.program_id(0)

    @pl.when(i == 0)
    def _():
        gsum_ref[...] = jnp.zeros_like(gsum_ref)
        cnt_ref[...] = jnp.zeros_like(cnt_ref)

    h = jax.nn.relu(s_ref[...] * (a0_ref[...] + a1_ref[...] + hs_ref[...])
                    + b_ref[...])
    seg = batch_ref[...]
    gid = lax.broadcasted_iota(jnp.int32, (G, BLK), 0)
    oh = (seg == gid).astype(jnp.float32)
    gsum_ref[...] += jnp.dot(oh, h, preferred_element_type=jnp.float32)
    cnt_ref[...] += jnp.sum(oh, axis=1, keepdims=True)

    @pl.when(i == NBLK - 1)
    def _():
        g = gsum_ref[...] / jnp.maximum(cnt_ref[...], 1.0)
        a = jax.nn.relu(jnp.dot(g, wm1_ref[...],
                                preferred_element_type=jnp.float32)
                        + bm1_ref[...])
        o = jnp.dot(a, wm2_ref[...],
                    preferred_element_type=jnp.float32) + bm2_ref[...]
        o = o - jnp.max(o, axis=1, keepdims=True)
        eo = jnp.exp(o)
        o_ref[...] = eo / jnp.sum(eo, axis=1, keepdims=True)


def _final_layer(accp, hs, s, b, batch3d, wm1, bm1, wm2, bm2):
    return pl.pallas_call(
        _final_body,
        grid=(NBLK,),
        in_specs=[
            pl.BlockSpec((None, BLK, HID), lambda i: (0, i, 0)),
            pl.BlockSpec((None, BLK, HID), lambda i: (1, i, 0)),
            pl.BlockSpec((BLK, HID), lambda i: (i, 0)),
            pl.BlockSpec((BLK, 1), lambda i: (i, 0)),
            pl.BlockSpec((1, HID), lambda i: (0, 0)),
            pl.BlockSpec((None, 1, BLK), lambda i: (i, 0, 0)),
            pl.BlockSpec((HID, HID // 2), lambda i: (0, 0)),
            pl.BlockSpec((1, HID // 2), lambda i: (0, 0)),
            pl.BlockSpec((HID // 2, 2), lambda i: (0, 0)),
            pl.BlockSpec((1, 2), lambda i: (0, 0)),
        ],
        out_specs=pl.BlockSpec((G, 2), lambda i: (0, 0)),
        out_shape=jax.ShapeDtypeStruct((G, 2), jnp.float32),
        scratch_shapes=[
            pltpu.VMEM((G, HID), jnp.float32),
            pltpu.VMEM((G, 1), jnp.float32),
        ],
    )(accp, accp, hs, s, b, batch3d, wm1, bm1, wm2, bm2)


def kernel(x, edge_index, edge_attr, batch, emb, W1, b1, W2, b2,
           Wm1, bm1, Wm2, bm2):
    src, dst = edge_index[0], edge_index[1]

    spread = (jnp.arange(EP - E, dtype=jnp.int32) * 37) % N
    srcp = jnp.concatenate([src, spread])
    dstp = jnp.concatenate([dst, spread])
    ewp = jnp.pad(edge_attr, (0, EP - E))

    xg = jnp.concatenate(
        [x, (jnp.arange(XGP - N, dtype=jnp.int32) * 37 + 11) % VOCAB])
    h0f, degp = _sc_gather_deg(emb, xg,
                               dstp.reshape(EP // ECHUNK, ECHUNK),
                               ewp.reshape(EP // ECHUNK, ECHUNK))
    h0p = h0f[:NP_]
    xp = jnp.pad(x, (0, NP_ - N)).reshape(NP_, 1)
    mask = (xp != 0).astype(jnp.float32)

    hs1, dinv = _prep_layer(h0p, mask, degp.reshape(NSC, NP_, 1), W1)

    accp1 = _sc_aggregate(hs1, srcp, dstp, ewp)
    hs2 = _combine_layer(accp1, hs1, dinv, b1.reshape(1, HID), W2)

    accp2 = _sc_aggregate(hs2, srcp, dstp, ewp)

    batchp = jnp.pad(batch, (0, NP_ - N), constant_values=-1)
    batch3d = batchp.reshape(NBLK, 1, BLK)

    return _final_layer(accp2, hs2, dinv, b2.reshape(1, HID), batch3d,
                        Wm1, bm1.reshape(1, HID // 2), Wm2,
                        bm2.reshape(1, 2))


# speedup vs baseline: 15.7698x; 1.1028x over previous
import functools

import jax
import jax.numpy as jnp
from jax import lax
from jax.experimental import pallas as pl
from jax.experimental.pallas import tpu as pltpu
from jax.experimental.pallas import tpu_sc as plsc

N = 10000
E = 320000
G = 128
VOCAB = 100000
EMB = 128
HID = 128

NP_ = 10240
BLK = 1024
NBLK = NP_ // BLK

NSC = 2
NTILE = 16
ECHUNK = 128
NCHUNK = 80
EP = NSC * NTILE * NCHUNK * ECHUNK
EPT = NCHUNK * ECHUNK
RPT = NP_ // NTILE


def _agg_body(hs_hbm, src_hbm, de_hbm, ew_hbm, out_hbm,
              srcv0, srcv1, dstv0, dstv1, ewv0, ewv1, rows0, rows1,
              acc_sh, sem0, sem1, sc0, sc1):
    c = lax.axis_index("c")
    s = lax.axis_index("s")
    tid = c * NTILE + s
    tb = tid * EPT

    def zero_body(t, _):
        r = t // 8
        j = t % 8
        rows0[r, pl.ds(j * 16, 16)] = jnp.zeros((16,), jnp.float32)
        return 0
    lax.fori_loop(0, ECHUNK * 8, zero_body, 0, unroll=8)
    for k in range(RPT // ECHUNK):
        pltpu.sync_copy(rows0, acc_sh.at[pl.ds(s * RPT + k * ECHUNK, ECHUNK)])
    plsc.subcore_barrier()

    def start_gather(sref, rows, sem, i):
        pltpu.sync_copy(src_hbm.at[pl.ds(tb + i * ECHUNK, ECHUNK)], sref)
        pltpu.async_copy(hs_hbm.at[sref], rows, sem)

    def scale_and_scatter(rows, ewv, dstv, sc, i):
        pltpu.sync_copy(de_hbm.at[pl.ds(tb + i * ECHUNK, ECHUNK)], dstv)
        pltpu.sync_copy(ew_hbm.at[pl.ds(tb + i * ECHUNK, ECHUNK)], ewv)

        def scale_body(g, _):
            ewg = ewv[pl.ds(g * 16, 16)]
            for l in range(16):
                spl = jnp.full((16,), ewg[l], jnp.float32)
                for j in range(8):
                    rows[g * 16 + l, pl.ds(j * 16, 16)] = \
                        rows[g * 16 + l, pl.ds(j * 16, 16)] * spl
            return 0
        lax.fori_loop(0, ECHUNK // 16, scale_body, 0)
        pltpu.async_copy(rows, acc_sh.at[dstv], sc, add=True)

    def wait_gather(sref, rows, sem):
        pltpu.make_async_copy(hs_hbm.at[sref], rows, sem).wait()

    def wait_scatter(rows, dstv, sc):
        pltpu.make_async_copy(rows, acc_sh.at[dstv], sc).wait()

    start_gather(srcv0, rows0, sem0, 0)
    start_gather(srcv1, rows1, sem1, 1)

    def pair_body(p, _):
        i0 = p * 2
        wait_gather(srcv0, rows0, sem0)
        scale_and_scatter(rows0, ewv0, dstv0, sc0, i0)
        wait_gather(srcv1, rows1, sem1)
        scale_and_scatter(rows1, ewv1, dstv1, sc1, i0 + 1)
        wait_scatter(rows0, dstv0, sc0)
        start_gather(srcv0, rows0, sem0, i0 + 2)
        wait_scatter(rows1, dstv1, sc1)
        start_gather(srcv1, rows1, sem1, i0 + 3)
        return 0
    lax.fori_loop(0, NCHUNK // 2 - 1, pair_body, 0)

    wait_gather(srcv0, rows0, sem0)
    scale_and_scatter(rows0, ewv0, dstv0, sc0, NCHUNK - 2)
    wait_gather(srcv1, rows1, sem1)
    scale_and_scatter(rows1, ewv1, dstv1, sc1, NCHUNK - 1)
    wait_scatter(rows0, dstv0, sc0)
    wait_scatter(rows1, dstv1, sc1)

    plsc.subcore_barrier()
    pltpu.sync_copy(acc_sh.at[pl.ds(s * RPT, RPT)],
                    out_hbm.at[c, pl.ds(s * RPT, RPT)])


def _sc_aggregate(hs, srcp, dstp, ewp):
    mesh = plsc.VectorSubcoreMesh(core_axis_name="c", subcore_axis_name="s")
    f = pl.kernel(
        _agg_body,
        out_type=jax.ShapeDtypeStruct((NSC, NP_, HID), jnp.float32),
        mesh=mesh,
        scratch_types=[
            pltpu.VMEM((ECHUNK,), jnp.int32),
            pltpu.VMEM((ECHUNK,), jnp.int32),
            pltpu.VMEM((ECHUNK,), jnp.int32),
            pltpu.VMEM((ECHUNK,), jnp.int32),
            pltpu.VMEM((ECHUNK,), jnp.float32),
            pltpu.VMEM((ECHUNK,), jnp.float32),
            pltpu.VMEM((ECHUNK, HID), jnp.float32),
            pltpu.VMEM((ECHUNK, HID), jnp.float32),
            pltpu.VMEM_SHARED((NP_, HID), jnp.float32),
            pltpu.SemaphoreType.DMA,
            pltpu.SemaphoreType.DMA,
            pltpu.SemaphoreType.DMA,
            pltpu.SemaphoreType.DMA,
        ],
    )
    return f(hs, srcp, dstp, ewp)


GPT = NP_ // (NSC * NTILE)


def _gd_body(emb_hbm, xg_hbm, dst2_hbm, ew2_hbm, h0_hbm, deg_hbm,
             idxv, idxv2, rows_v, rows_v2, dst_t, ew_t, zv, deg_sh, sem):
    c = lax.axis_index("c")
    s = lax.axis_index("s")
    wid = c * NTILE + s

    pltpu.sync_copy(dst2_hbm.at[pl.ds(wid * NCHUNK, NCHUNK)], dst_t)
    pltpu.sync_copy(ew2_hbm.at[pl.ds(wid * NCHUNK, NCHUNK)], ew_t)

    def zero_body(t, _):
        zv[pl.ds(t * 16, 16)] = jnp.zeros((16,), jnp.float32)
        return 0
    lax.fori_loop(0, RPT // 16, zero_body, 0, unroll=8)
    pltpu.sync_copy(zv, deg_sh.at[pl.ds(s * RPT, RPT)])
    plsc.subcore_barrier()

    for k in range(2):
        base = wid * GPT + k * ECHUNK
        pltpu.sync_copy(xg_hbm.at[pl.ds(base, ECHUNK)], idxv)
        pltpu.async_copy(emb_hbm.at[idxv], rows_v, sem).wait()
        pltpu.sync_copy(rows_v, h0_hbm.at[pl.ds(base, ECHUNK)])
    base = wid * GPT + 2 * ECHUNK
    pltpu.sync_copy(xg_hbm.at[pl.ds(base, 64)], idxv2)
    pltpu.async_copy(emb_hbm.at[idxv2], rows_v2, sem).wait()
    pltpu.sync_copy(rows_v2, h0_hbm.at[pl.ds(base, 64)])

    def deg_body(i, _):
        pltpu.sync_copy(ew_t.at[i], deg_sh.at[dst_t.at[i]], add=True)
        return 0
    lax.fori_loop(0, NCHUNK, deg_body, 0)

    plsc.subcore_barrier()
    pltpu.sync_copy(deg_sh.at[pl.ds(s * RPT, RPT)],
                    deg_hbm.at[c, pl.ds(s * RPT, RPT)])


def _sc_gather_deg(emb, xg, dst2, ew2):
    mesh = plsc.VectorSubcoreMesh(core_axis_name="c", subcore_axis_name="s")
    f = pl.kernel(
        _gd_body,
        out_type=(jax.ShapeDtypeStruct((NP_, EMB), jnp.float32),
                  jax.ShapeDtypeStruct((NSC, NP_), jnp.float32)),
        mesh=mesh,
        scratch_types=[
            pltpu.VMEM((ECHUNK,), jnp.int32),
            pltpu.VMEM((64,), jnp.int32),
            pltpu.VMEM((ECHUNK, EMB), jnp.float32),
            pltpu.VMEM((64, EMB), jnp.float32),
            pltpu.VMEM((NCHUNK, ECHUNK), jnp.int32),
            pltpu.VMEM((NCHUNK, ECHUNK), jnp.float32),
            pltpu.VMEM((RPT,), jnp.float32),
            pltpu.VMEM_SHARED((NP_,), jnp.float32),
            pltpu.SemaphoreType.DMA,
        ],
    )
    return f(emb, xg, dst2, ew2)


def _prep_body(h_ref, m_ref, d0_ref, d1_ref, w_ref, o_ref, s_ref):
    dv = lax.rsqrt(d0_ref[...] + d1_ref[...] + 1.0)
    s_ref[...] = dv
    h = h_ref[...] * m_ref[...]
    o_ref[...] = dv * jnp.dot(h, w_ref[...],
                              preferred_element_type=jnp.float32)


def _prep_layer(h, m, degp3, w):
    return pl.pallas_call(
        _prep_body,
        grid=(NBLK,),
        in_specs=[
            pl.BlockSpec((BLK, HID), lambda i: (i, 0)),
            pl.BlockSpec((BLK, 1), lambda i: (i, 0)),
            pl.BlockSpec((None, BLK, 1), lambda i: (0, i, 0)),
            pl.BlockSpec((None, BLK, 1), lambda i: (1, i, 0)),
            pl.BlockSpec((HID, HID), lambda i: (0, 0)),
        ],
        out_specs=[
            pl.BlockSpec((BLK, HID), lambda i: (i, 0)),
            pl.BlockSpec((BLK, 1), lambda i: (i, 0)),
        ],
        out_shape=[
            jax.ShapeDtypeStruct((NP_, HID), jnp.float32),
            jax.ShapeDtypeStruct((NP_, 1), jnp.float32),
        ],
    )(h, m, degp3, degp3, w)


def _combine_body(a0_ref, a1_ref, hs_ref, s_ref, b_ref, w_ref, o_ref):
    t = jax.nn.relu(s_ref[...] * (a0_ref[...] + a1_ref[...] + hs_ref[...])
                    + b_ref[...])
    o_ref[...] = s_ref[...] * jnp.dot(t, w_ref[...],
                                      preferred_element_type=jnp.float32)


def _combine_layer(accp, hs, s, b, w):
    return pl.pallas_call(
        _combine_body,
        grid=(NBLK,),
        in_specs=[
            pl.BlockSpec((None, BLK, HID), lambda i: (0, i, 0)),
            pl.BlockSpec((None, BLK, HID), lambda i: (1, i, 0)),
            pl.BlockSpec((BLK, HID), lambda i: (i, 0)),
            pl.BlockSpec((BLK, 1), lambda i: (i, 0)),
            pl.BlockSpec((1, HID), lambda i: (0, 0)),
            pl.BlockSpec((HID, HID), lambda i: (0, 0)),
        ],
        out_specs=pl.BlockSpec((BLK, HID), lambda i: (i, 0)),
        out_shape=jax.ShapeDtypeStruct((NP_, HID), jnp.float32),
    )(accp, accp, hs, s, b, w)


def _final_body(a0_ref, a1_ref, hs_ref, s_ref, b_ref, batch_ref,
                wm1_ref, bm1_ref, wm2_ref, bm2_ref, o_ref,
                gsum_ref, cnt_ref):
    i = pl.program_id(0)

    @pl.when(i == 0)
    def _():
        gsum_ref[...] = jnp.zeros_like(gsum_ref)
        cnt_ref[...] = jnp.zeros_like(cnt_ref)

    h = jax.nn.relu(s_ref[...] * (a0_ref[...] + a1_ref[...] + hs_ref[...])
                    + b_ref[...])
    seg = batch_ref[...]
    gid = lax.broadcasted_iota(jnp.int32, (G, BLK), 0)
    oh = (seg == gid).astype(jnp.float32)
    gsum_ref[...] += jnp.dot(oh, h, preferred_element_type=jnp.float32)
    cnt_ref[...] += jnp.sum(oh, axis=1, keepdims=True)

    @pl.when(i == NBLK - 1)
    def _():
        g = gsum_ref[...] / jnp.maximum(cnt_ref[...], 1.0)
        a = jax.nn.relu(jnp.dot(g, wm1_ref[...],
                                preferred_element_type=jnp.float32)
                        + bm1_ref[...])
        o = jnp.dot(a, wm2_ref[...],
                    preferred_element_type=jnp.float32) + bm2_ref[...]
        o = o - jnp.max(o, axis=1, keepdims=True)
        eo = jnp.exp(o)
        o_ref[...] = eo / jnp.sum(eo, axis=1, keepdims=True)


def _final_layer(accp, hs, s, b, batch3d, wm1, bm1, wm2, bm2):
    return pl.pallas_call(
        _final_body,
        grid=(NBLK,),
        in_specs=[
            pl.BlockSpec((None, BLK, HID), lambda i: (0, i, 0)),
            pl.BlockSpec((None, BLK, HID), lambda i: (1, i, 0)),
            pl.BlockSpec((BLK, HID), lambda i: (i, 0)),
            pl.BlockSpec((BLK, 1), lambda i: (i, 0)),
            pl.BlockSpec((1, HID), lambda i: (0, 0)),
            pl.BlockSpec((None, 1, BLK), lambda i: (i, 0, 0)),
            pl.BlockSpec((HID, HID // 2), lambda i: (0, 0)),
            pl.BlockSpec((1, HID // 2), lambda i: (0, 0)),
            pl.BlockSpec((HID // 2, 2), lambda i: (0, 0)),
            pl.BlockSpec((1, 2), lambda i: (0, 0)),
        ],
        out_specs=pl.BlockSpec((G, 2), lambda i: (0, 0)),
        out_shape=jax.ShapeDtypeStruct((G, 2), jnp.float32),
        scratch_shapes=[
            pltpu.VMEM((G, HID), jnp.float32),
            pltpu.VMEM((G, 1), jnp.float32),
        ],
    )(accp, accp, hs, s, b, batch3d, wm1, bm1, wm2, bm2)


def kernel(x, edge_index, edge_attr, batch, emb, W1, b1, W2, b2,
           Wm1, bm1, Wm2, bm2):
    src, dst = edge_index[0], edge_index[1]

    spread = (jnp.arange(EP - E, dtype=jnp.int32) * 37) % N
    srcp = jnp.concatenate([src, spread])
    dstp = jnp.concatenate([dst, spread])
    ewp = jnp.pad(edge_attr, (0, EP - E))

    xg = jnp.concatenate(
        [x, (jnp.arange(NP_ - N, dtype=jnp.int32) * 37 + 11) % VOCAB])
    h0p, degp = _sc_gather_deg(emb, xg,
                               dstp.reshape(EP // ECHUNK, ECHUNK),
                               ewp.reshape(EP // ECHUNK, ECHUNK))
    xp = jnp.pad(x, (0, NP_ - N)).reshape(NP_, 1)
    mask = (xp != 0).astype(jnp.float32)

    hs1, dinv = _prep_layer(h0p, mask, degp.reshape(NSC, NP_, 1), W1)

    accp1 = _sc_aggregate(hs1, srcp, dstp, ewp)
    hs2 = _combine_layer(accp1, hs1, dinv, b1.reshape(1, HID), W2)

    accp2 = _sc_aggregate(hs2, srcp, dstp, ewp)

    batchp = jnp.pad(batch, (0, NP_ - N), constant_values=-1)
    batch3d = batchp.reshape(NBLK, 1, BLK)

    return _final_layer(accp2, hs2, dinv, b2.reshape(1, HID), batch3d,
                        Wm1, bm1.reshape(1, HID // 2), Wm2,
                        bm2.reshape(1, 2))


# speedup vs baseline: 19.8729x; 1.2602x over previous
import functools

import jax
import jax.numpy as jnp
from jax import lax
from jax.experimental import pallas as pl
from jax.experimental.pallas import tpu as pltpu
from jax.experimental.pallas import tpu_sc as plsc

N = 10000
E = 320000
G = 128
VOCAB = 100000
EMB = 128
HID = 128

NP_ = 10240
BLK = 1024
NBLK = NP_ // BLK

NSC = 2
NTILE = 16
ECHUNK = 128
NCHUNK = 80
EP = NSC * NTILE * NCHUNK * ECHUNK
EPT = NCHUNK * ECHUNK
RPT = NP_ // NTILE


def _agg_body(hs_hbm, src_hbm, de_hbm, ew_hbm, out_hbm,
              srcv0, srcv1, dstv0, dstv1, ewv0, ewv1, rows0, rows1,
              acc_sh, sem0, sem1, sc0, sc1, se0, se1):
    c = lax.axis_index("c")
    s = lax.axis_index("s")
    tid = c * NTILE + s
    tb = tid * EPT

    def zero_body(t, _):
        r = t // 8
        j = t % 8
        rows0[r, pl.ds(j * 16, 16)] = jnp.zeros((16,), jnp.float32)
        return 0
    lax.fori_loop(0, ECHUNK * 8, zero_body, 0, unroll=8)
    for k in range(RPT // ECHUNK):
        pltpu.sync_copy(rows0, acc_sh.at[pl.ds(s * RPT + k * ECHUNK, ECHUNK)])
    plsc.subcore_barrier()

    def start_chunk(sref, dstv, ewv, rows, sem, se, i):
        pltpu.sync_copy(src_hbm.at[pl.ds(tb + i * ECHUNK, ECHUNK)], sref)
        pltpu.async_copy(hs_hbm.at[sref], rows, sem)
        pltpu.async_copy(de_hbm.at[pl.ds(tb + i * ECHUNK, ECHUNK)], dstv, se)
        pltpu.async_copy(ew_hbm.at[pl.ds(tb + i * ECHUNK, ECHUNK)], ewv, se)

    def scale_and_scatter(rows, ewv, dstv, se, sc, i):
        pltpu.make_async_copy(
            de_hbm.at[pl.ds(tb + i * ECHUNK, ECHUNK)], dstv, se).wait()
        pltpu.make_async_copy(
            ew_hbm.at[pl.ds(tb + i * ECHUNK, ECHUNK)], ewv, se).wait()

        def scale_body(g, _):
            ewg = ewv[pl.ds(g * 16, 16)]
            for l in range(16):
                spl = jnp.full((16,), ewg[l], jnp.float32)
                for j in range(8):
                    rows[g * 16 + l, pl.ds(j * 16, 16)] = \
                        rows[g * 16 + l, pl.ds(j * 16, 16)] * spl
            return 0
        lax.fori_loop(0, ECHUNK // 16, scale_body, 0)
        pltpu.async_copy(rows, acc_sh.at[dstv], sc, add=True)

    def wait_gather(sref, rows, sem):
        pltpu.make_async_copy(hs_hbm.at[sref], rows, sem).wait()

    def wait_scatter(rows, dstv, sc):
        pltpu.make_async_copy(rows, acc_sh.at[dstv], sc).wait()

    start_chunk(srcv0, dstv0, ewv0, rows0, sem0, se0, 0)
    start_chunk(srcv1, dstv1, ewv1, rows1, sem1, se1, 1)

    def pair_body(p, _):
        i0 = p * 2
        wait_gather(srcv0, rows0, sem0)
        scale_and_scatter(rows0, ewv0, dstv0, se0, sc0, i0)
        wait_gather(srcv1, rows1, sem1)
        scale_and_scatter(rows1, ewv1, dstv1, se1, sc1, i0 + 1)
        wait_scatter(rows0, dstv0, sc0)
        start_chunk(srcv0, dstv0, ewv0, rows0, sem0, se0, i0 + 2)
        wait_scatter(rows1, dstv1, sc1)
        start_chunk(srcv1, dstv1, ewv1, rows1, sem1, se1, i0 + 3)
        return 0
    lax.fori_loop(0, NCHUNK // 2 - 1, pair_body, 0)

    wait_gather(srcv0, rows0, sem0)
    scale_and_scatter(rows0, ewv0, dstv0, se0, sc0, NCHUNK - 2)
    wait_gather(srcv1, rows1, sem1)
    scale_and_scatter(rows1, ewv1, dstv1, se1, sc1, NCHUNK - 1)
    wait_scatter(rows0, dstv0, sc0)
    wait_scatter(rows1, dstv1, sc1)

    plsc.subcore_barrier()
    pltpu.sync_copy(acc_sh.at[pl.ds(s * RPT, RPT)],
                    out_hbm.at[c, pl.ds(s * RPT, RPT)])


def _sc_aggregate(hs, srcp, dstp, ewp):
    mesh = plsc.VectorSubcoreMesh(core_axis_name="c", subcore_axis_name="s")
    f = pl.kernel(
        _agg_body,
        out_type=jax.ShapeDtypeStruct((NSC, NP_, HID), jnp.float32),
        mesh=mesh,
        scratch_types=[
            pltpu.VMEM((ECHUNK,), jnp.int32),
            pltpu.VMEM((ECHUNK,), jnp.int32),
            pltpu.VMEM((ECHUNK,), jnp.int32),
            pltpu.VMEM((ECHUNK,), jnp.int32),
            pltpu.VMEM((ECHUNK,), jnp.float32),
            pltpu.VMEM((ECHUNK,), jnp.float32),
            pltpu.VMEM((ECHUNK, HID), jnp.float32),
            pltpu.VMEM((ECHUNK, HID), jnp.float32),
            pltpu.VMEM_SHARED((NP_, HID), jnp.float32),
            pltpu.SemaphoreType.DMA,
            pltpu.SemaphoreType.DMA,
            pltpu.SemaphoreType.DMA,
            pltpu.SemaphoreType.DMA,
            pltpu.SemaphoreType.DMA,
            pltpu.SemaphoreType.DMA,
        ],
    )
    return f(hs, srcp, dstp, ewp)


GPT = NP_ // (NSC * NTILE)


def _gd_body(emb_hbm, xg_hbm, dst2_hbm, ew2_hbm, h0_hbm, deg_hbm,
             idxv, idxv2, rows_v, rows_v2, dst_t, ew_t, zv, deg_sh, sem):
    c = lax.axis_index("c")
    s = lax.axis_index("s")
    wid = c * NTILE + s

    pltpu.sync_copy(dst2_hbm.at[pl.ds(wid * NCHUNK, NCHUNK)], dst_t)
    pltpu.sync_copy(ew2_hbm.at[pl.ds(wid * NCHUNK, NCHUNK)], ew_t)

    def zero_body(t, _):
        zv[pl.ds(t * 16, 16)] = jnp.zeros((16,), jnp.float32)
        return 0
    lax.fori_loop(0, RPT // 16, zero_body, 0, unroll=8)
    pltpu.sync_copy(zv, deg_sh.at[pl.ds(s * RPT, RPT)])
    plsc.subcore_barrier()

    for k in range(2):
        base = wid * GPT + k * ECHUNK
        pltpu.sync_copy(xg_hbm.at[pl.ds(base, ECHUNK)], idxv)
        pltpu.async_copy(emb_hbm.at[idxv], rows_v, sem).wait()
        pltpu.sync_copy(rows_v, h0_hbm.at[pl.ds(base, ECHUNK)])
    base = wid * GPT + 2 * ECHUNK
    pltpu.sync_copy(xg_hbm.at[pl.ds(base, 64)], idxv2)
    pltpu.async_copy(emb_hbm.at[idxv2], rows_v2, sem).wait()
    pltpu.sync_copy(rows_v2, h0_hbm.at[pl.ds(base, 64)])

    def deg_body(i, _):
        pltpu.sync_copy(ew_t.at[i], deg_sh.at[dst_t.at[i]], add=True)
        return 0
    lax.fori_loop(0, NCHUNK, deg_body, 0)

    plsc.subcore_barrier()
    pltpu.sync_copy(deg_sh.at[pl.ds(s * RPT, RPT)],
                    deg_hbm.at[c, pl.ds(s * RPT, RPT)])


def _sc_gather_deg(emb, xg, dst2, ew2):
    mesh = plsc.VectorSubcoreMesh(core_axis_name="c", subcore_axis_name="s")
    f = pl.kernel(
        _gd_body,
        out_type=(jax.ShapeDtypeStruct((NP_, EMB), jnp.float32),
                  jax.ShapeDtypeStruct((NSC, NP_), jnp.float32)),
        mesh=mesh,
        scratch_types=[
            pltpu.VMEM((ECHUNK,), jnp.int32),
            pltpu.VMEM((64,), jnp.int32),
            pltpu.VMEM((ECHUNK, EMB), jnp.float32),
            pltpu.VMEM((64, EMB), jnp.float32),
            pltpu.VMEM((NCHUNK, ECHUNK), jnp.int32),
            pltpu.VMEM((NCHUNK, ECHUNK), jnp.float32),
            pltpu.VMEM((RPT,), jnp.float32),
            pltpu.VMEM_SHARED((NP_,), jnp.float32),
            pltpu.SemaphoreType.DMA,
        ],
    )
    return f(emb, xg, dst2, ew2)


def _prep_body(h_ref, m_ref, d0_ref, d1_ref, w_ref, o_ref, s_ref):
    dv = lax.rsqrt(d0_ref[...] + d1_ref[...] + 1.0)
    s_ref[...] = dv
    h = h_ref[...] * m_ref[...]
    o_ref[...] = dv * jnp.dot(h, w_ref[...],
                              preferred_element_type=jnp.float32)


def _prep_layer(h, m, degp3, w):
    return pl.pallas_call(
        _prep_body,
        grid=(NBLK,),
        in_specs=[
            pl.BlockSpec((BLK, HID), lambda i: (i, 0)),
            pl.BlockSpec((BLK, 1), lambda i: (i, 0)),
            pl.BlockSpec((None, BLK, 1), lambda i: (0, i, 0)),
            pl.BlockSpec((None, BLK, 1), lambda i: (1, i, 0)),
            pl.BlockSpec((HID, HID), lambda i: (0, 0)),
        ],
        out_specs=[
            pl.BlockSpec((BLK, HID), lambda i: (i, 0)),
            pl.BlockSpec((BLK, 1), lambda i: (i, 0)),
        ],
        out_shape=[
            jax.ShapeDtypeStruct((NP_, HID), jnp.float32),
            jax.ShapeDtypeStruct((NP_, 1), jnp.float32),
        ],
    )(h, m, degp3, degp3, w)


def _combine_body(a0_ref, a1_ref, hs_ref, s_ref, b_ref, w_ref, o_ref):
    t = jax.nn.relu(s_ref[...] * (a0_ref[...] + a1_ref[...] + hs_ref[...])
                    + b_ref[...])
    o_ref[...] = s_ref[...] * jnp.dot(t, w_ref[...],
                                      preferred_element_type=jnp.float32)


def _combine_layer(accp, hs, s, b, w):
    return pl.pallas_call(
        _combine_body,
        grid=(NBLK,),
        in_specs=[
            pl.BlockSpec((None, BLK, HID), lambda i: (0, i, 0)),
            pl.BlockSpec((None, BLK, HID), lambda i: (1, i, 0)),
            pl.BlockSpec((BLK, HID), lambda i: (i, 0)),
            pl.BlockSpec((BLK, 1), lambda i: (i, 0)),
            pl.BlockSpec((1, HID), lambda i: (0, 0)),
            pl.BlockSpec((HID, HID), lambda i: (0, 0)),
        ],
        out_specs=pl.BlockSpec((BLK, HID), lambda i: (i, 0)),
        out_shape=jax.ShapeDtypeStruct((NP_, HID), jnp.float32),
    )(accp, accp, hs, s, b, w)


def _final_body(a0_ref, a1_ref, hs_ref, s_ref, b_ref, batch_ref,
                wm1_ref, bm1_ref, wm2_ref, bm2_ref, o_ref,
                gsum_ref, cnt_ref):
    i = pl.program_id(0)

    @pl.when(i == 0)
    def _():
        gsum_ref[...] = jnp.zeros_like(gsum_ref)
        cnt_ref[...] = jnp.zeros_like(cnt_ref)

    h = jax.nn.relu(s_ref[...] * (a0_ref[...] + a1_ref[...] + hs_ref[...])
                    + b_ref[...])
    seg = batch_ref[...]
    gid = lax.broadcasted_iota(jnp.int32, (G, BLK), 0)
    oh = (seg == gid).astype(jnp.float32)
    gsum_ref[...] += jnp.dot(oh, h, preferred_element_type=jnp.float32)
    cnt_ref[...] += jnp.sum(oh, axis=1, keepdims=True)

    @pl.when(i == NBLK - 1)
    def _():
        g = gsum_ref[...] / jnp.maximum(cnt_ref[...], 1.0)
        a = jax.nn.relu(jnp.dot(g, wm1_ref[...],
                                preferred_element_type=jnp.float32)
                        + bm1_ref[...])
        o = jnp.dot(a, wm2_ref[...],
                    preferred_element_type=jnp.float32) + bm2_ref[...]
        o = o - jnp.max(o, axis=1, keepdims=True)
        eo = jnp.exp(o)
        o_ref[...] = eo / jnp.sum(eo, axis=1, keepdims=True)


def _final_layer(accp, hs, s, b, batch3d, wm1, bm1, wm2, bm2):
    return pl.pallas_call(
        _final_body,
        grid=(NBLK,),
        in_specs=[
            pl.BlockSpec((None, BLK, HID), lambda i: (0, i, 0)),
            pl.BlockSpec((None, BLK, HID), lambda i: (1, i, 0)),
            pl.BlockSpec((BLK, HID), lambda i: (i, 0)),
            pl.BlockSpec((BLK, 1), lambda i: (i, 0)),
            pl.BlockSpec((1, HID), lambda i: (0, 0)),
            pl.BlockSpec((None, 1, BLK), lambda i: (i, 0, 0)),
            pl.BlockSpec((HID, HID // 2), lambda i: (0, 0)),
            pl.BlockSpec((1, HID // 2), lambda i: (0, 0)),
            pl.BlockSpec((HID // 2, 2), lambda i: (0, 0)),
            pl.BlockSpec((1, 2), lambda i: (0, 0)),
        ],
        out_specs=pl.BlockSpec((G, 2), lambda i: (0, 0)),
        out_shape=jax.ShapeDtypeStruct((G, 2), jnp.float32),
        scratch_shapes=[
            pltpu.VMEM((G, HID), jnp.float32),
            pltpu.VMEM((G, 1), jnp.float32),
        ],
    )(accp, accp, hs, s, b, batch3d, wm1, bm1, wm2, bm2)


def kernel(x, edge_index, edge_attr, batch, emb, W1, b1, W2, b2,
           Wm1, bm1, Wm2, bm2):
    src, dst = edge_index[0], edge_index[1]

    spread = (jnp.arange(EP - E, dtype=jnp.int32) * 37) % N
    srcp = jnp.concatenate([src, spread])
    dstp = jnp.concatenate([dst, spread])
    ewp = jnp.pad(edge_attr, (0, EP - E))

    xg = jnp.concatenate(
        [x, (jnp.arange(NP_ - N, dtype=jnp.int32) * 37 + 11) % VOCAB])
    h0p, degp = _sc_gather_deg(emb, xg,
                               dstp.reshape(EP // ECHUNK, ECHUNK),
                               ewp.reshape(EP // ECHUNK, ECHUNK))
    xp = jnp.pad(x, (0, NP_ - N)).reshape(NP_, 1)
    mask = (xp != 0).astype(jnp.float32)

    hs1, dinv = _prep_layer(h0p, mask, degp.reshape(NSC, NP_, 1), W1)

    accp1 = _sc_aggregate(hs1, srcp, dstp, ewp)
    hs2 = _combine_layer(accp1, hs1, dinv, b1.reshape(1, HID), W2)

    accp2 = _sc_aggregate(hs2, srcp, dstp, ewp)

    batchp = jnp.pad(batch, (0, NP_ - N), constant_values=-1)
    batch3d = batchp.reshape(NBLK, 1, BLK)

    return _final_layer(accp2, hs2, dinv, b2.reshape(1, HID), batch3d,
                        Wm1, bm1.reshape(1, HID // 2), Wm2,
                        bm2.reshape(1, 2))


# speedup vs baseline: 23.1593x; 1.1654x over previous
import functools

import jax
import jax.numpy as jnp
from jax import lax
from jax.experimental import pallas as pl
from jax.experimental.pallas import tpu as pltpu
from jax.experimental.pallas import tpu_sc as plsc

N = 10000
E = 320000
G = 128
VOCAB = 100000
EMB = 128
HID = 128

NP_ = 10240
BLK = 1024
NBLK = NP_ // BLK

NSC = 2
NTILE = 16
ECHUNK = 128
NCHUNK = 80
EP = NSC * NTILE * NCHUNK * ECHUNK
EPT = NCHUNK * ECHUNK
RPT = NP_ // NTILE


def _agg_body(hs_hbm, src_hbm, de_hbm, ew_hbm, out_hbm,
              srcA0, srcA1, srcB0, srcB1, dstv0, dstv1, ewv0, ewv1,
              rows0, rows1, acc_sh, sem0, sem1, sc0, sc1, se0, se1,
              spA0, spA1, spB0, spB1):
    c = lax.axis_index("c")
    s = lax.axis_index("s")
    tid = c * NTILE + s
    tb = tid * EPT

    def zero_body(t, _):
        r = t // 8
        j = t % 8
        rows0[r, pl.ds(j * 16, 16)] = jnp.zeros((16,), jnp.float32)
        return 0
    lax.fori_loop(0, ECHUNK * 8, zero_body, 0, unroll=8)
    for k in range(RPT // ECHUNK):
        pltpu.sync_copy(rows0, acc_sh.at[pl.ds(s * RPT + k * ECHUNK, ECHUNK)])
    plsc.subcore_barrier()

    def prefetch_src(sref, sp, i):
        pltpu.async_copy(src_hbm.at[pl.ds(tb + i * ECHUNK, ECHUNK)], sref, sp)

    def start_chunk(sref, sp, dstv, ewv, rows, sem, se, i):
        pltpu.make_async_copy(
            src_hbm.at[pl.ds(tb + i * ECHUNK, ECHUNK)], sref, sp).wait()
        pltpu.async_copy(hs_hbm.at[sref], rows, sem)
        pltpu.async_copy(de_hbm.at[pl.ds(tb + i * ECHUNK, ECHUNK)], dstv, se)
        pltpu.async_copy(ew_hbm.at[pl.ds(tb + i * ECHUNK, ECHUNK)], ewv, se)

    def scale_and_scatter(rows, ewv, dstv, se, sc, i):
        pltpu.make_async_copy(
            de_hbm.at[pl.ds(tb + i * ECHUNK, ECHUNK)], dstv, se).wait()
        pltpu.make_async_copy(
            ew_hbm.at[pl.ds(tb + i * ECHUNK, ECHUNK)], ewv, se).wait()

        def scale_body(g, _):
            ewg = ewv[pl.ds(g * 16, 16)]
            for l in range(16):
                spl = jnp.full((16,), ewg[l], jnp.float32)
                for j in range(8):
                    rows[g * 16 + l, pl.ds(j * 16, 16)] = \
                        rows[g * 16 + l, pl.ds(j * 16, 16)] * spl
            return 0
        lax.fori_loop(0, ECHUNK // 16, scale_body, 0)
        pltpu.async_copy(rows, acc_sh.at[dstv], sc, add=True)

    def wait_gather(sref, rows, sem):
        pltpu.make_async_copy(hs_hbm.at[sref], rows, sem).wait()

    def wait_scatter(rows, dstv, sc):
        pltpu.make_async_copy(rows, acc_sh.at[dstv], sc).wait()

    prefetch_src(srcA0, spA0, 0)
    prefetch_src(srcA1, spA1, 1)
    start_chunk(srcA0, spA0, dstv0, ewv0, rows0, sem0, se0, 0)
    start_chunk(srcA1, spA1, dstv1, ewv1, rows1, sem1, se1, 1)
    prefetch_src(srcB0, spB0, 2)
    prefetch_src(srcB1, spB1, 3)

    def quad_body(q, _):
        i0 = q * 4
        wait_gather(srcA0, rows0, sem0)
        scale_and_scatter(rows0, ewv0, dstv0, se0, sc0, i0)
        wait_scatter(rows0, dstv0, sc0)
        prefetch_src(srcA0, spA0, i0 + 4)
        start_chunk(srcB0, spB0, dstv0, ewv0, rows0, sem0, se0, i0 + 2)
        wait_gather(srcA1, rows1, sem1)
        scale_and_scatter(rows1, ewv1, dstv1, se1, sc1, i0 + 1)
        wait_scatter(rows1, dstv1, sc1)
        prefetch_src(srcA1, spA1, i0 + 5)
        start_chunk(srcB1, spB1, dstv1, ewv1, rows1, sem1, se1, i0 + 3)
        wait_gather(srcB0, rows0, sem0)
        scale_and_scatter(rows0, ewv0, dstv0, se0, sc0, i0 + 2)
        wait_scatter(rows0, dstv0, sc0)
        prefetch_src(srcB0, spB0, i0 + 6)
        start_chunk(srcA0, spA0, dstv0, ewv0, rows0, sem0, se0, i0 + 4)
        wait_gather(srcB1, rows1, sem1)
        scale_and_scatter(rows1, ewv1, dstv1, se1, sc1, i0 + 3)
        wait_scatter(rows1, dstv1, sc1)
        prefetch_src(srcB1, spB1, i0 + 7)
        start_chunk(srcA1, spA1, dstv1, ewv1, rows1, sem1, se1, i0 + 5)
        return 0
    lax.fori_loop(0, NCHUNK // 4 - 1, quad_body, 0)

    wait_gather(srcA0, rows0, sem0)
    scale_and_scatter(rows0, ewv0, dstv0, se0, sc0, NCHUNK - 4)
    wait_scatter(rows0, dstv0, sc0)
    start_chunk(srcB0, spB0, dstv0, ewv0, rows0, sem0, se0, NCHUNK - 2)
    wait_gather(srcA1, rows1, sem1)
    scale_and_scatter(rows1, ewv1, dstv1, se1, sc1, NCHUNK - 3)
    wait_scatter(rows1, dstv1, sc1)
    start_chunk(srcB1, spB1, dstv1, ewv1, rows1, sem1, se1, NCHUNK - 1)
    wait_gather(srcB0, rows0, sem0)
    scale_and_scatter(rows0, ewv0, dstv0, se0, sc0, NCHUNK - 2)
    wait_gather(srcB1, rows1, sem1)
    scale_and_scatter(rows1, ewv1, dstv1, se1, sc1, NCHUNK - 1)
    wait_scatter(rows0, dstv0, sc0)
    wait_scatter(rows1, dstv1, sc1)

    plsc.subcore_barrier()
    pltpu.sync_copy(acc_sh.at[pl.ds(s * RPT, RPT)],
                    out_hbm.at[c, pl.ds(s * RPT, RPT)])


def _sc_aggregate(hs, srcp, dstp, ewp):
    mesh = plsc.VectorSubcoreMesh(core_axis_name="c", subcore_axis_name="s")
    f = pl.kernel(
        _agg_body,
        out_type=jax.ShapeDtypeStruct((NSC, NP_, HID), jnp.float32),
        mesh=mesh,
        scratch_types=[
            pltpu.VMEM((ECHUNK,), jnp.int32),
            pltpu.VMEM((ECHUNK,), jnp.int32),
            pltpu.VMEM((ECHUNK,), jnp.int32),
            pltpu.VMEM((ECHUNK,), jnp.int32),
            pltpu.VMEM((ECHUNK,), jnp.int32),
            pltpu.VMEM((ECHUNK,), jnp.int32),
            pltpu.VMEM((ECHUNK,), jnp.float32),
            pltpu.VMEM((ECHUNK,), jnp.float32),
            pltpu.VMEM((ECHUNK, HID), jnp.float32),
            pltpu.VMEM((ECHUNK, HID), jnp.float32),
            pltpu.VMEM_SHARED((NP_, HID), jnp.float32),
            pltpu.SemaphoreType.DMA,
            pltpu.SemaphoreType.DMA,
            pltpu.SemaphoreType.DMA,
            pltpu.SemaphoreType.DMA,
            pltpu.SemaphoreType.DMA,
            pltpu.SemaphoreType.DMA,
            pltpu.SemaphoreType.DMA,
            pltpu.SemaphoreType.DMA,
            pltpu.SemaphoreType.DMA,
            pltpu.SemaphoreType.DMA,
        ],
    )
    return f(hs, srcp, dstp, ewp)


GPT = NP_ // (NSC * NTILE)


def _gd_body(emb_hbm, xg_hbm, dst2_hbm, ew2_hbm, h0_hbm, deg_hbm,
             idxv, idxv2, rows_v, rows_v2, dst_t, ew_t, zv, deg_sh, sem):
    c = lax.axis_index("c")
    s = lax.axis_index("s")
    wid = c * NTILE + s

    pltpu.sync_copy(dst2_hbm.at[pl.ds(wid * NCHUNK, NCHUNK)], dst_t)
    pltpu.sync_copy(ew2_hbm.at[pl.ds(wid * NCHUNK, NCHUNK)], ew_t)

    def zero_body(t, _):
        zv[pl.ds(t * 16, 16)] = jnp.zeros((16,), jnp.float32)
        return 0
    lax.fori_loop(0, RPT // 16, zero_body, 0, unroll=8)
    pltpu.sync_copy(zv, deg_sh.at[pl.ds(s * RPT, RPT)])
    plsc.subcore_barrier()

    for k in range(2):
        base = wid * GPT + k * ECHUNK
        pltpu.sync_copy(xg_hbm.at[pl.ds(base, ECHUNK)], idxv)
        pltpu.async_copy(emb_hbm.at[idxv], rows_v, sem).wait()
        pltpu.sync_copy(rows_v, h0_hbm.at[pl.ds(base, ECHUNK)])
    base = wid * GPT + 2 * ECHUNK
    pltpu.sync_copy(xg_hbm.at[pl.ds(base, 64)], idxv2)
    pltpu.async_copy(emb_hbm.at[idxv2], rows_v2, sem).wait()
    pltpu.sync_copy(rows_v2, h0_hbm.at[pl.ds(base, 64)])

    def deg_body(i, _):
        pltpu.sync_copy(ew_t.at[i], deg_sh.at[dst_t.at[i]], add=True)
        return 0
    lax.fori_loop(0, NCHUNK, deg_body, 0)

    plsc.subcore_barrier()
    pltpu.sync_copy(deg_sh.at[pl.ds(s * RPT, RPT)],
                    deg_hbm.at[c, pl.ds(s * RPT, RPT)])


def _sc_gather_deg(emb, xg, dst2, ew2):
    mesh = plsc.VectorSubcoreMesh(core_axis_name="c", subcore_axis_name="s")
    f = pl.kernel(
        _gd_body,
        out_type=(jax.ShapeDtypeStruct((NP_, EMB), jnp.float32),
                  jax.ShapeDtypeStruct((NSC, NP_), jnp.float32)),
        mesh=mesh,
        scratch_types=[
            pltpu.VMEM((ECHUNK,), jnp.int32),
            pltpu.VMEM((64,), jnp.int32),
            pltpu.VMEM((ECHUNK, EMB), jnp.float32),
            pltpu.VMEM((64, EMB), jnp.float32),
            pltpu.VMEM((NCHUNK, ECHUNK), jnp.int32),
            pltpu.VMEM((NCHUNK, ECHUNK), jnp.float32),
            pltpu.VMEM((RPT,), jnp.float32),
            pltpu.VMEM_SHARED((NP_,), jnp.float32),
            pltpu.SemaphoreType.DMA,
        ],
    )
    return f(emb, xg, dst2, ew2)


def _prep_body(h_ref, m_ref, d0_ref, d1_ref, w_ref, o_ref, s_ref):
    dv = lax.rsqrt(d0_ref[...] + d1_ref[...] + 1.0)
    s_ref[...] = dv
    h = h_ref[...] * m_ref[...]
    o_ref[...] = dv * jnp.dot(h, w_ref[...],
                              preferred_element_type=jnp.float32)


def _prep_layer(h, m, degp3, w):
    return pl.pallas_call(
        _prep_body,
        grid=(NBLK,),
        in_specs=[
            pl.BlockSpec((BLK, HID), lambda i: (i, 0)),
            pl.BlockSpec((BLK, 1), lambda i: (i, 0)),
            pl.BlockSpec((None, BLK, 1), lambda i: (0, i, 0)),
            pl.BlockSpec((None, BLK, 1), lambda i: (1, i, 0)),
            pl.BlockSpec((HID, HID), lambda i: (0, 0)),
        ],
        out_specs=[
            pl.BlockSpec((BLK, HID), lambda i: (i, 0)),
            pl.BlockSpec((BLK, 1), lambda i: (i, 0)),
        ],
        out_shape=[
            jax.ShapeDtypeStruct((NP_, HID), jnp.float32),
            jax.ShapeDtypeStruct((NP_, 1), jnp.float32),
        ],
    )(h, m, degp3, degp3, w)


def _combine_body(a0_ref, a1_ref, hs_ref, s_ref, b_ref, w_ref, o_ref):
    t = jax.nn.relu(s_ref[...] * (a0_ref[...] + a1_ref[...] + hs_ref[...])
                    + b_ref[...])
    o_ref[...] = s_ref[...] * jnp.dot(t, w_ref[...],
                                      preferred_element_type=jnp.float32)


def _combine_layer(accp, hs, s, b, w):
    return pl.pallas_call(
        _combine_body,
        grid=(NBLK,),
        in_specs=[
            pl.BlockSpec((None, BLK, HID), lambda i: (0, i, 0)),
            pl.BlockSpec((None, BLK, HID), lambda i: (1, i, 0)),
            pl.BlockSpec((BLK, HID), lambda i: (i, 0)),
            pl.BlockSpec((BLK, 1), lambda i: (i, 0)),
            pl.BlockSpec((1, HID), lambda i: (0, 0)),
            pl.BlockSpec((HID, HID), lambda i: (0, 0)),
        ],
        out_specs=pl.BlockSpec((BLK, HID), lambda i: (i, 0)),
        out_shape=jax.ShapeDtypeStruct((NP_, HID), jnp.float32),
    )(accp, accp, hs, s, b, w)


def _final_body(a0_ref, a1_ref, hs_ref, s_ref, b_ref, batch_ref,
                wm1_ref, bm1_ref, wm2_ref, bm2_ref, o_ref,
                gsum_ref, cnt_ref):
    i = pl.program_id(0)

    @pl.when(i == 0)
    def _():
        gsum_ref[...] = jnp.zeros_like(gsum_ref)
        cnt_ref[...] = jnp.zeros_like(cnt_ref)

    h = jax.nn.relu(s_ref[...] * (a0_ref[...] + a1_ref[...] + hs_ref[...])
                    + b_ref[...])
    seg = batch_ref[...]
    gid = lax.broadcasted_iota(jnp.int32, (G, BLK), 0)
    oh = (seg == gid).astype(jnp.float32)
    gsum_ref[...] += jnp.dot(oh, h, preferred_element_type=jnp.float32)
    cnt_ref[...] += jnp.sum(oh, axis=1, keepdims=True)

    @pl.when(i == NBLK - 1)
    def _():
        g = gsum_ref[...] / jnp.maximum(cnt_ref[...], 1.0)
        a = jax.nn.relu(jnp.dot(g, wm1_ref[...],
                                preferred_element_type=jnp.float32)
                        + bm1_ref[...])
        o = jnp.dot(a, wm2_ref[...],
                    preferred_element_type=jnp.float32) + bm2_ref[...]
        o = o - jnp.max(o, axis=1, keepdims=True)
        eo = jnp.exp(o)
        o_ref[...] = eo / jnp.sum(eo, axis=1, keepdims=True)


def _final_layer(accp, hs, s, b, batch3d, wm1, bm1, wm2, bm2):
    return pl.pallas_call(
        _final_body,
        grid=(NBLK,),
        in_specs=[
            pl.BlockSpec((None, BLK, HID), lambda i: (0, i, 0)),
            pl.BlockSpec((None, BLK, HID), lambda i: (1, i, 0)),
            pl.BlockSpec((BLK, HID), lambda i: (i, 0)),
            pl.BlockSpec((BLK, 1), lambda i: (i, 0)),
            pl.BlockSpec((1, HID), lambda i: (0, 0)),
            pl.BlockSpec((None, 1, BLK), lambda i: (i, 0, 0)),
            pl.BlockSpec((HID, HID // 2), lambda i: (0, 0)),
            pl.BlockSpec((1, HID // 2), lambda i: (0, 0)),
            pl.BlockSpec((HID // 2, 2), lambda i: (0, 0)),
            pl.BlockSpec((1, 2), lambda i: (0, 0)),
        ],
        out_specs=pl.BlockSpec((G, 2), lambda i: (0, 0)),
        out_shape=jax.ShapeDtypeStruct((G, 2), jnp.float32),
        scratch_shapes=[
            pltpu.VMEM((G, HID), jnp.float32),
            pltpu.VMEM((G, 1), jnp.float32),
        ],
    )(accp, accp, hs, s, b, batch3d, wm1, bm1, wm2, bm2)


def kernel(x, edge_index, edge_attr, batch, emb, W1, b1, W2, b2,
           Wm1, bm1, Wm2, bm2):
    src, dst = edge_index[0], edge_index[1]

    spread = (jnp.arange(EP - E, dtype=jnp.int32) * 37) % N
    srcp = jnp.concatenate([src, spread])
    dstp = jnp.concatenate([dst, spread])
    ewp = jnp.pad(edge_attr, (0, EP - E))

    xg = jnp.concatenate(
        [x, (jnp.arange(NP_ - N, dtype=jnp.int32) * 37 + 11) % VOCAB])
    h0p, degp = _sc_gather_deg(emb, xg,
                               dstp.reshape(EP // ECHUNK, ECHUNK),
                               ewp.reshape(EP // ECHUNK, ECHUNK))
    xp = jnp.pad(x, (0, NP_ - N)).reshape(NP_, 1)
    mask = (xp != 0).astype(jnp.float32)

    hs1, dinv = _prep_layer(h0p, mask, degp.reshape(NSC, NP_, 1), W1)

    accp1 = _sc_aggregate(hs1, srcp, dstp, ewp)
    hs2 = _combine_layer(accp1, hs1, dinv, b1.reshape(1, HID), W2)

    accp2 = _sc_aggregate(hs2, srcp, dstp, ewp)

    batchp = jnp.pad(batch, (0, NP_ - N), constant_values=-1)
    batch3d = batchp.reshape(NBLK, 1, BLK)

    return _final_layer(accp2, hs2, dinv, b2.reshape(1, HID), batch3d,
                        Wm1, bm1.reshape(1, HID // 2), Wm2,
                        bm2.reshape(1, 2))
